# baseline (speedup 1.0000x reference)
import jax
import jax.numpy as jnp
from jax import lax
from jax.experimental import pallas as pl
from jax.experimental.pallas import tpu as pltpu
from jax.experimental.pallas import tpu_sc as plsc

N = 20000
C = 81
CP = 128
G = 64
NS = 16
NT = 1280
NPAD = NS * NT
GRPS = NT // 16
POS_K = 32
NEG_K = 96
TOT_K = 128
REG_W = 2.0

_MESH = plsc.VectorSubcoreMesh(
    core_axis_name="c", subcore_axis_name="s", num_cores=1, num_subcores=NS
)


def _ln(x):
    bits = lax.bitcast_convert_type(x, jnp.int32)
    e = jnp.right_shift(bits, 23) & 0xFF
    m = lax.bitcast_convert_type((bits & 0x7FFFFF) | 0x3F800000, jnp.float32)
    big = m > 1.4142135623730951
    m = jnp.where(big, m * 0.5, m)
    ef = (e - 127 + big.astype(jnp.int32)).astype(jnp.float32)
    s = (m - 1.0) / (m + 1.0)
    s2 = s * s
    p = jnp.float32(1.0 / 9.0)
    p = p * s2 + jnp.float32(1.0 / 7.0)
    p = p * s2 + jnp.float32(0.2)
    p = p * s2 + jnp.float32(1.0 / 3.0)
    p = p * s2 + jnp.float32(1.0)
    lnm = 2.0 * s * p
    return ef * jnp.float32(0.6931471805599453) + lnm


def _body(rois_hbm, scores_hbm, txty_hbm, gt_hbm, gtl_hbm, out_hbm,
          rois_l, txty_l, gt_l, gtl_l,
          v_l, hist_l, histall_l, gsum_l, cdf_l,
          sel_l, gidx_l, rows2_l, rows_l, part_l, partall_l, out_l,
          sh_hist, sh_part, sem):
    wid = lax.axis_index("s")
    base = wid * NT
    iota = lax.iota(jnp.int32, 16)
    zc = jnp.zeros((16,), jnp.int32)
    zf = jnp.zeros((16,), jnp.float32)

    pltpu.sync_copy(rois_hbm.at[pl.ds(base * 4, NT * 4)], rois_l)
    pltpu.sync_copy(txty_hbm.at[pl.ds(base * 4, NT * 4)], txty_l)
    pltpu.sync_copy(gt_hbm, gt_l)
    pltpu.sync_copy(gtl_hbm, gtl_l)

    for q in range(4):
        hist_l[pl.ds(q * 16, 16)] = zc

    def group_body(g, _):
        ridx = (g * 16 + iota) * 4
        ax0 = plsc.load_gather(rois_l, [ridx])
        ay0 = plsc.load_gather(rois_l, [ridx + 1])
        ax1 = plsc.load_gather(rois_l, [ridx + 2])
        ay1 = plsc.load_gather(rois_l, [ridx + 3])
        area_a = (ax1 - ax0) * (ay1 - ay0)

        def gt_body(jj, carry):
            best, bidx = carry
            for u in range(4):
                j = jj * 4 + u
                bx0 = plsc.load_gather(gt_l, [zc + j * 4])
                by0 = plsc.load_gather(gt_l, [zc + (j * 4 + 1)])
                bx1 = plsc.load_gather(gt_l, [zc + (j * 4 + 2)])
                by1 = plsc.load_gather(gt_l, [zc + (j * 4 + 3)])
                ab = (bx1 - bx0) * (by1 - by0)
                w = jnp.maximum(jnp.minimum(ax1, bx1) - jnp.maximum(ax0, bx0), 0.0)
                h = jnp.maximum(jnp.minimum(ay1, by1) - jnp.maximum(ay0, by0), 0.0)
                inter = w * h
                iou = inter / (area_a + ab - inter + 1e-8)
                upd = iou > best
                best = jnp.where(upd, iou, best)
                bidx = jnp.where(upd, j, bidx)
            return best, bidx

        best, bidx = lax.fori_loop(
            0, 16, gt_body, (jnp.full((16,), -1.0, jnp.float32), zc)
        )
        v_l[pl.ds(g * 16, 16)] = bidx
        valid = (base + g * 16 + iota) < N
        plsc.addupdate_scatter(hist_l, [bidx], zc + 1, mask=valid)
        return 0

    lax.fori_loop(0, GRPS, group_body, 0)

    pltpu.sync_copy(hist_l, sh_hist.at[pl.ds(wid * G, G)])
    plsc.subcore_barrier()
    pltpu.sync_copy(sh_hist, histall_l)

    gq = []
    for q in range(4):
        acc = zc
        for w in range(NS):
            acc = acc + histall_l[pl.ds(w * G + q * 16, 16)]
        gsum_l[pl.ds(q * 16, 16)] = acc
        gq.append(acc)

    cq = []
    tot = jnp.int32(0)
    for q in range(4):
        c = plsc.cumsum(gq[q]) + tot
        tot = tot + jnp.sum(gq[q])
        cdf_l[pl.ds(q * 16, 16)] = c
        cq.append(c)

    npos = zc
    nneg = zc
    for q in range(4):
        cprev = cq[q] - gq[q]
        npos = npos + plsc.all_reduce_population_count(cprev <= N - POS_K)
        nneg = nneg + plsc.all_reduce_population_count(cq[q] < NEG_K)
    tpos = jnp.max(npos) - 1
    tneg = jnp.max(nneg)
    cpos = jnp.max(plsc.load_gather(cdf_l, [zc + tpos]))
    rpos = POS_K - (N - cpos)
    cneg = jnp.max(plsc.load_gather(cdf_l, [zc + tneg]))
    gneg = jnp.max(plsc.load_gather(gsum_l, [zc + tneg]))
    rneg = NEG_K - (cneg - gneg)

    hp = plsc.load_gather(histall_l, [iota * G + tpos])
    hn = plsc.load_gather(histall_l, [iota * G + tneg])
    base_pos = jnp.sum(jnp.where(iota < wid, hp, 0))
    base_neg = jnp.sum(jnp.where(iota < wid, hn, 0))

    for q in range(8):
        sel_l[pl.ds(q * 16, 16)] = zc

    def sel_body(g, carry):
        cntv, tpv, tnv = carry
        v = v_l[pl.ds(g * 16, 16)]
        valid = (base + g * 16 + iota) < N
        hi = jnp.logical_and(v > tpos, valid)
        mtp = jnp.logical_and(v == tpos, valid)
        rkp = tpv + plsc.cumsum(mtp.astype(jnp.int32)) - 1 + base_pos
        ptie = jnp.logical_and(mtp, rkp < rpos)
        lo = jnp.logical_and(v < tneg, valid)
        mtn = jnp.logical_and(v == tneg, valid)
        rkn = tnv + plsc.cumsum(mtn.astype(jnp.int32)) - 1 + base_neg
        ntie = jnp.logical_and(mtn, rkn < rneg)
        sel = jnp.logical_or(jnp.logical_or(hi, ptie), jnp.logical_or(lo, ntie))
        pos = cntv + plsc.cumsum(sel.astype(jnp.int32)) - 1
        plsc.store_scatter(sel_l, [pos], g * 16 + iota, mask=sel)
        cntv = cntv + plsc.all_reduce_population_count(sel)
        tpv = tpv + plsc.all_reduce_population_count(mtp)
        tnv = tnv + plsc.all_reduce_population_count(mtn)
        return cntv, tpv, tnv

    cntv, _, _ = lax.fori_loop(0, GRPS, sel_body, (zc, zc, zc))
    cnt_s = jnp.max(cntv)

    def gb_body(q, _):
        sl = pl.ds(q * 16, 16)
        gidx_l[sl] = sel_l[sl] + base
        return 0

    lax.fori_loop(0, 8, gb_body, 0)
    pltpu.async_copy(scores_hbm.at[gidx_l], rows2_l, sem).wait()

    def flat_body(r, _):
        for k in range(CP // 16):
            rows_l[pl.ds(r * CP + k * 16, 16)] = rows2_l[r, pl.ds(k * 16, 16)]
        return 0

    lax.fori_loop(0, cnt_s, flat_body, 0)

    ngrp = jnp.right_shift(cnt_s + 15, 4)

    def loss_body(q, carry):
        cls_acc, reg_acc = carry
        rvec = q * 16 + iota
        actf = (rvec < cnt_s).astype(jnp.float32)
        lidx = plsc.load_gather(sel_l, [rvec])
        vr = plsc.load_gather(v_l, [lidx])
        lab = (vr >= 1).astype(jnp.int32)
        labf = lab.astype(jnp.float32)
        gl = plsc.load_gather(gtl_l, [vr])
        t = jnp.clip(gl * lab, 0, C - 1)
        rbase = rvec * CP

        def mx_body(cc, m):
            for u in range(4):
                x = plsc.load_gather(rows_l, [rbase + (cc * 4 + u)])
                m = jnp.maximum(m, x)
            return m

        m = lax.fori_loop(0, CP // 4, mx_body, jnp.full((16,), -3e38, jnp.float32))

        def se_body(cc, se):
            for u in range(4):
                x = plsc.load_gather(rows_l, [rbase + (cc * 4 + u)])
                se = se + jnp.exp(x - m)
            return se

        se = lax.fori_loop(0, CP // 4, se_body, zf)
        lse = m + _ln(se)
        logit_t = plsc.load_gather(rows_l, [rbase + t])
        cls_acc = cls_acc + (logit_t - lse) * actf

        l4 = lidx * 4
        v4 = vr * 4
        ax0 = plsc.load_gather(rois_l, [l4])
        ay0 = plsc.load_gather(rois_l, [l4 + 1])
        ax1 = plsc.load_gather(rois_l, [l4 + 2])
        ay1 = plsc.load_gather(rois_l, [l4 + 3])
        p0 = plsc.load_gather(txty_l, [l4])
        p1 = plsc.load_gather(txty_l, [l4 + 1])
        p2 = plsc.load_gather(txty_l, [l4 + 2])
        p3 = plsc.load_gather(txty_l, [l4 + 3])
        gx0 = plsc.load_gather(gt_l, [v4])
        gy0 = plsc.load_gather(gt_l, [v4 + 1])
        gx1 = plsc.load_gather(gt_l, [v4 + 2])
        gy1 = plsc.load_gather(gt_l, [v4 + 3])
        aw = ax1 - ax0
        ah = ay1 - ay0
        axc = ax0 + 0.5 * aw
        ayc = ay0 + 0.5 * ah
        gw = gx1 - gx0
        gh = gy1 - gy0
        gxc = gx0 + 0.5 * gw
        gyc = gy0 + 0.5 * gh
        awm = jnp.maximum(aw, 1e-8)
        ahm = jnp.maximum(ah, 1e-8)
        tx = (gxc - axc) / awm
        ty = (gyc - ayc) / ahm
        tw = _ln(jnp.maximum(gw, 1e-8) / awm)
        th = _ln(jnp.maximum(gh, 1e-8) / ahm)
        sl1 = zf
        for d in (labf * (p0 - tx), labf * (p1 - ty),
                  labf * (p2 - tw), labf * (p3 - th)):
            ad = jnp.abs(d)
            sl1 = sl1 + jnp.where(ad < 1.0, 0.5 * ad * ad, ad - 0.5)
        reg_acc = reg_acc + sl1 * actf
        return cls_acc, reg_acc

    cls_acc, reg_acc = lax.fori_loop(0, ngrp, loss_body, (zf, zf))

    part_l[pl.ds(0, 16)] = cls_acc
    part_l[pl.ds(16, 16)] = reg_acc
    pltpu.sync_copy(part_l, sh_part.at[pl.ds(wid * 32, 32)])
    plsc.subcore_barrier()

    @pl.when(wid == 0)
    def _():
        pltpu.sync_copy(sh_part, partall_l)
        cs = zf
        rs = zf
        for w in range(NS):
            cs = cs + partall_l[pl.ds(w * 32, 16)]
            rs = rs + partall_l[pl.ds(w * 32 + 16, 16)]
        cls_t = jnp.sum(cs)
        reg_t = jnp.sum(rs)
        outv = jnp.where(iota == 0, -cls_t * (1.0 / TOT_K),
                         jnp.where(iota == 1, (REG_W / TOT_K) * reg_t, 0.0))
        out_l[...] = outv
        pltpu.sync_copy(out_l, out_hbm)


_sc_call = pl.kernel(
    _body,
    out_type=jax.ShapeDtypeStruct((16,), jnp.float32),
    mesh=_MESH,
    compiler_params=pltpu.CompilerParams(needs_layout_passes=False),
    scratch_types=[
        pltpu.VMEM((NT * 4,), jnp.float32),
        pltpu.VMEM((NT * 4,), jnp.float32),
        pltpu.VMEM((G * 4,), jnp.float32),
        pltpu.VMEM((G,), jnp.int32),
        pltpu.VMEM((NT,), jnp.int32),
        pltpu.VMEM((G,), jnp.int32),
        pltpu.VMEM((NS * G,), jnp.int32),
        pltpu.VMEM((G,), jnp.int32),
        pltpu.VMEM((G,), jnp.int32),
        pltpu.VMEM((TOT_K,), jnp.int32),
        pltpu.VMEM((TOT_K,), jnp.int32),
        pltpu.VMEM((TOT_K, CP), jnp.float32),
        pltpu.VMEM((TOT_K * CP,), jnp.float32),
        pltpu.VMEM((32,), jnp.float32),
        pltpu.VMEM((NS * 32,), jnp.float32),
        pltpu.VMEM((16,), jnp.float32),
        pltpu.VMEM_SHARED((NS * G,), jnp.int32),
        pltpu.VMEM_SHARED((NS * 32,), jnp.float32),
        pltpu.SemaphoreType.DMA,
    ],
)


def kernel(rpn_proposals_bboxes, roi_score, roi_bboxes_txtytwth, gt_bboxes, gt_labels):
    rois_p = jnp.pad(rpn_proposals_bboxes, ((0, NPAD - N), (0, 0))).reshape(-1)
    txty_p = jnp.pad(roi_bboxes_txtytwth, ((0, NPAD - N), (0, 0))).reshape(-1)
    scores_p = jnp.pad(roi_score, ((0, 0), (0, CP - C)), constant_values=-1e30)
    out = _sc_call(rois_p, scores_p, txty_p, gt_bboxes.reshape(-1),
                   gt_labels.astype(jnp.int32))
    return out[0], out[1]


# speedup vs baseline: 1.5581x; 1.5581x over previous
import jax
import jax.numpy as jnp
from jax import lax
from jax.experimental import pallas as pl
from jax.experimental.pallas import tpu as pltpu
from jax.experimental.pallas import tpu_sc as plsc

N = 20000
C = 81
G = 64
NSA = 32
NTA = 640
GRPSA = NTA // 16
NSB = 16
NTB = 1280
GRPSB = NTB // 16
NPAD = NSA * NTA
POS_K = 32
NEG_K = 96
TOT_K = 128
REG_W = 2.0

_MESH_A = plsc.VectorSubcoreMesh(
    core_axis_name="c", subcore_axis_name="s", num_cores=2, num_subcores=16
)
_MESH_B = plsc.VectorSubcoreMesh(
    core_axis_name="c", subcore_axis_name="s", num_cores=1, num_subcores=16
)
_SC_PARAMS = pltpu.CompilerParams(needs_layout_passes=False)


def _ln(x):
    bits = lax.bitcast_convert_type(x, jnp.int32)
    e = jnp.right_shift(bits, 23) & 0xFF
    m = lax.bitcast_convert_type((bits & 0x7FFFFF) | 0x3F800000, jnp.float32)
    big = m > 1.4142135623730951
    m = jnp.where(big, m * 0.5, m)
    ef = (e - 127 + big.astype(jnp.int32)).astype(jnp.float32)
    s = (m - 1.0) / (m + 1.0)
    s2 = s * s
    p = jnp.float32(1.0 / 9.0)
    p = p * s2 + jnp.float32(1.0 / 7.0)
    p = p * s2 + jnp.float32(0.2)
    p = p * s2 + jnp.float32(1.0 / 3.0)
    p = p * s2 + jnp.float32(1.0)
    lnm = 2.0 * s * p
    return ef * jnp.float32(0.6931471805599453) + lnm


def _body_a(rois_hbm, gt_hbm, v_hbm, hist_hbm, rois_l, gt_l, v_l, hist_l):
    wid = lax.axis_index("s") * 2 + lax.axis_index("c")
    base = wid * NTA
    iota = lax.iota(jnp.int32, 16)
    zc = jnp.zeros((16,), jnp.int32)

    pltpu.sync_copy(rois_hbm.at[pl.ds(base * 4, NTA * 4)], rois_l)
    pltpu.sync_copy(gt_hbm, gt_l)

    for q in range(4):
        hist_l[pl.ds(q * 16, 16)] = zc

    def group_body(g, _):
        ridx = (g * 16 + iota) * 4
        ax0 = plsc.load_gather(rois_l, [ridx])
        ay0 = plsc.load_gather(rois_l, [ridx + 1])
        ax1 = plsc.load_gather(rois_l, [ridx + 2])
        ay1 = plsc.load_gather(rois_l, [ridx + 3])
        area_a = (ax1 - ax0) * (ay1 - ay0)

        def gt_body(jj, carry):
            best, bidx = carry
            for u in range(4):
                j = jj * 4 + u
                bx0 = plsc.load_gather(gt_l, [zc + j * 4])
                by0 = plsc.load_gather(gt_l, [zc + (j * 4 + 1)])
                bx1 = plsc.load_gather(gt_l, [zc + (j * 4 + 2)])
                by1 = plsc.load_gather(gt_l, [zc + (j * 4 + 3)])
                ab = (bx1 - bx0) * (by1 - by0)
                w = jnp.maximum(jnp.minimum(ax1, bx1) - jnp.maximum(ax0, bx0), 0.0)
                h = jnp.maximum(jnp.minimum(ay1, by1) - jnp.maximum(ay0, by0), 0.0)
                inter = w * h
                iou = inter / (area_a + ab - inter + 1e-8)
                upd = iou > best
                best = jnp.where(upd, iou, best)
                bidx = jnp.where(upd, j, bidx)
            return best, bidx

        best, bidx = lax.fori_loop(
            0, 16, gt_body, (jnp.full((16,), -1.0, jnp.float32), zc)
        )
        v_l[pl.ds(g * 16, 16)] = bidx
        valid = (base + g * 16 + iota) < N
        plsc.addupdate_scatter(hist_l, [bidx], zc + 1, mask=valid)
        return 0

    lax.fori_loop(0, GRPSA, group_body, 0)
    pltpu.sync_copy(v_l, v_hbm.at[pl.ds(base, NTA)])
    pltpu.sync_copy(hist_l, hist_hbm.at[pl.ds(wid * G, G)])


_call_a = pl.kernel(
    _body_a,
    out_type=(
        jax.ShapeDtypeStruct((NPAD,), jnp.int32),
        jax.ShapeDtypeStruct((NSA * G,), jnp.int32),
    ),
    mesh=_MESH_A,
    compiler_params=_SC_PARAMS,
    scratch_types=[
        pltpu.VMEM((NTA * 4,), jnp.float32),
        pltpu.VMEM((G * 4,), jnp.float32),
        pltpu.VMEM((NTA,), jnp.int32),
        pltpu.VMEM((G,), jnp.int32),
    ],
)


def _body_b(rois_hbm, txty_hbm, gt_hbm, gtl_hbm, v_hbm, hist_hbm,
            gidx_hbm, tcls_hbm, regv_hbm,
            rois_l, txty_l, gt_l, gtl_l, v_l, histall_l, gsum_l, cdf_l,
            sel_l, gidx_l, tcls_l, pos_l, cntst_l, cntall_l, part_l,
            partall_l, out_l,
            sh_gidx, sh_tcls, sh_cnt, sh_part):
    wid = lax.axis_index("s")
    base = wid * NTB
    iota = lax.iota(jnp.int32, 16)
    zc = jnp.zeros((16,), jnp.int32)
    zf = jnp.zeros((16,), jnp.float32)

    pltpu.sync_copy(rois_hbm.at[pl.ds(base * 4, NTB * 4)], rois_l)
    pltpu.sync_copy(txty_hbm.at[pl.ds(base * 4, NTB * 4)], txty_l)
    pltpu.sync_copy(gt_hbm, gt_l)
    pltpu.sync_copy(gtl_hbm, gtl_l)
    pltpu.sync_copy(v_hbm.at[pl.ds(base, NTB)], v_l)
    pltpu.sync_copy(hist_hbm, histall_l)

    gq = []
    for q in range(4):
        acc = zc
        for w in range(NSA):
            acc = acc + histall_l[pl.ds(w * G + q * 16, 16)]
        gsum_l[pl.ds(q * 16, 16)] = acc
        gq.append(acc)

    cq = []
    tot = jnp.int32(0)
    for q in range(4):
        cc = plsc.cumsum(gq[q]) + tot
        tot = tot + jnp.sum(gq[q])
        cdf_l[pl.ds(q * 16, 16)] = cc
        cq.append(cc)

    npos = zc
    nneg = zc
    for q in range(4):
        cprev = cq[q] - gq[q]
        npos = npos + plsc.all_reduce_population_count(cprev <= N - POS_K)
        nneg = nneg + plsc.all_reduce_population_count(cq[q] < NEG_K)
    tpos = jnp.max(npos) - 1
    tneg = jnp.max(nneg)
    cpos = jnp.max(plsc.load_gather(cdf_l, [zc + tpos]))
    rpos = POS_K - (N - cpos)
    cneg = jnp.max(plsc.load_gather(cdf_l, [zc + tneg]))
    gneg = jnp.max(plsc.load_gather(gsum_l, [zc + tneg]))
    rneg = NEG_K - (cneg - gneg)

    hp0 = plsc.load_gather(histall_l, [iota * G + tpos])
    hp1 = plsc.load_gather(histall_l, [(iota + 16) * G + tpos])
    hn0 = plsc.load_gather(histall_l, [iota * G + tneg])
    hn1 = plsc.load_gather(histall_l, [(iota + 16) * G + tneg])
    a2 = wid * 2
    base_pos = (jnp.sum(jnp.where(iota < a2, hp0, 0))
                + jnp.sum(jnp.where(iota + 16 < a2, hp1, 0)))
    base_neg = (jnp.sum(jnp.where(iota < a2, hn0, 0))
                + jnp.sum(jnp.where(iota + 16 < a2, hn1, 0)))

    for q in range(8):
        sel_l[pl.ds(q * 16, 16)] = zc

    def sel_body(g, carry):
        cntv, tpv, tnv = carry
        v = v_l[pl.ds(g * 16, 16)]
        valid = (base + g * 16 + iota) < N
        hi = jnp.logical_and(v > tpos, valid)
        mtp = jnp.logical_and(v == tpos, valid)
        rkp = tpv + plsc.cumsum(mtp.astype(jnp.int32)) - 1 + base_pos
        ptie = jnp.logical_and(mtp, rkp < rpos)
        lo = jnp.logical_and(v < tneg, valid)
        mtn = jnp.logical_and(v == tneg, valid)
        rkn = tnv + plsc.cumsum(mtn.astype(jnp.int32)) - 1 + base_neg
        ntie = jnp.logical_and(mtn, rkn < rneg)
        sel = jnp.logical_or(jnp.logical_or(hi, ptie), jnp.logical_or(lo, ntie))
        pos = cntv + plsc.cumsum(sel.astype(jnp.int32)) - 1
        plsc.store_scatter(sel_l, [pos], g * 16 + iota, mask=sel)
        cntv = cntv + plsc.all_reduce_population_count(sel)
        tpv = tpv + plsc.all_reduce_population_count(mtp)
        tnv = tnv + plsc.all_reduce_population_count(mtn)
        return cntv, tpv, tnv

    cntv, _, _ = lax.fori_loop(0, GRPSB, sel_body, (zc, zc, zc))
    cnt_s = jnp.max(cntv)

    ngrp = jnp.right_shift(cnt_s + 15, 4)

    def loss_body(q, reg_acc):
        rvec = q * 16 + iota
        actf = (rvec < cnt_s).astype(jnp.float32)
        lidx = plsc.load_gather(sel_l, [rvec])
        vr = plsc.load_gather(v_l, [lidx])
        lab = (vr >= 1).astype(jnp.int32)
        labf = lab.astype(jnp.float32)
        gl = plsc.load_gather(gtl_l, [vr])
        tcls_l[pl.ds(q * 16, 16)] = jnp.clip(gl * lab, 0, C - 1)
        gidx_l[pl.ds(q * 16, 16)] = lidx + base

        l4 = lidx * 4
        v4 = vr * 4
        ax0 = plsc.load_gather(rois_l, [l4])
        ay0 = plsc.load_gather(rois_l, [l4 + 1])
        ax1 = plsc.load_gather(rois_l, [l4 + 2])
        ay1 = plsc.load_gather(rois_l, [l4 + 3])
        p0 = plsc.load_gather(txty_l, [l4])
        p1 = plsc.load_gather(txty_l, [l4 + 1])
        p2 = plsc.load_gather(txty_l, [l4 + 2])
        p3 = plsc.load_gather(txty_l, [l4 + 3])
        gx0 = plsc.load_gather(gt_l, [v4])
        gy0 = plsc.load_gather(gt_l, [v4 + 1])
        gx1 = plsc.load_gather(gt_l, [v4 + 2])
        gy1 = plsc.load_gather(gt_l, [v4 + 3])
        aw = ax1 - ax0
        ah = ay1 - ay0
        axc = ax0 + 0.5 * aw
        ayc = ay0 + 0.5 * ah
        gw = gx1 - gx0
        gh = gy1 - gy0
        gxc = gx0 + 0.5 * gw
        gyc = gy0 + 0.5 * gh
        awm = jnp.maximum(aw, 1e-8)
        ahm = jnp.maximum(ah, 1e-8)
        tx = (gxc - axc) / awm
        ty = (gyc - ayc) / ahm
        tw = _ln(jnp.maximum(gw, 1e-8) / awm)
        th = _ln(jnp.maximum(gh, 1e-8) / ahm)
        sl1 = zf
        for d in (labf * (p0 - tx), labf * (p1 - ty),
                  labf * (p2 - tw), labf * (p3 - th)):
            ad = jnp.abs(d)
            sl1 = sl1 + jnp.where(ad < 1.0, 0.5 * ad * ad, ad - 0.5)
        return reg_acc + sl1 * actf

    reg_acc = lax.fori_loop(0, ngrp, loss_body, zf)

    cntst_l[...] = cntv
    pltpu.sync_copy(cntst_l, sh_cnt.at[pl.ds(wid * 16, 16)])
    part_l[...] = reg_acc
    pltpu.sync_copy(part_l, sh_part.at[pl.ds(wid * 16, 16)])
    plsc.subcore_barrier()
    pltpu.sync_copy(sh_cnt, cntall_l)
    cnts = plsc.load_gather(cntall_l, [iota * 16])
    offset = jnp.sum(jnp.where(iota < wid, cnts, 0))

    def pos_body(q, _):
        rr = q * 16 + iota
        pos_l[pl.ds(q * 16, 16)] = jnp.where(rr < cnt_s, offset + rr,
                                             TOT_K + rr)
        return 0

    lax.fori_loop(0, 8, pos_body, 0)
    pltpu.sync_copy(gidx_l, sh_gidx.at[pos_l])
    pltpu.sync_copy(tcls_l, sh_tcls.at[pos_l])
    plsc.subcore_barrier()

    @pl.when(wid == 0)
    def _():
        pltpu.sync_copy(sh_gidx.at[pl.ds(0, TOT_K)], gidx_hbm)
        pltpu.sync_copy(sh_tcls.at[pl.ds(0, TOT_K)], tcls_hbm)
        pltpu.sync_copy(sh_part, partall_l)
        rs = zf
        for w in range(NSB):
            rs = rs + partall_l[pl.ds(w * 16, 16)]
        reg_t = jnp.sum(rs)
        out_l[...] = jnp.where(iota == 0, (REG_W / TOT_K) * reg_t, 0.0)
        pltpu.sync_copy(out_l, regv_hbm)


_call_b = pl.kernel(
    _body_b,
    out_type=(
        jax.ShapeDtypeStruct((TOT_K,), jnp.int32),
        jax.ShapeDtypeStruct((TOT_K,), jnp.int32),
        jax.ShapeDtypeStruct((16,), jnp.float32),
    ),
    mesh=_MESH_B,
    compiler_params=_SC_PARAMS,
    scratch_types=[
        pltpu.VMEM((NTB * 4,), jnp.float32),
        pltpu.VMEM((NTB * 4,), jnp.float32),
        pltpu.VMEM((G * 4,), jnp.float32),
        pltpu.VMEM((G,), jnp.int32),
        pltpu.VMEM((NTB,), jnp.int32),
        pltpu.VMEM((NSA * G,), jnp.int32),
        pltpu.VMEM((G,), jnp.int32),
        pltpu.VMEM((G,), jnp.int32),
        pltpu.VMEM((TOT_K,), jnp.int32),
        pltpu.VMEM((TOT_K,), jnp.int32),
        pltpu.VMEM((TOT_K,), jnp.int32),
        pltpu.VMEM((TOT_K,), jnp.int32),
        pltpu.VMEM((16,), jnp.int32),
        pltpu.VMEM((NSB * 16,), jnp.int32),
        pltpu.VMEM((16,), jnp.float32),
        pltpu.VMEM((NSB * 16,), jnp.float32),
        pltpu.VMEM((16,), jnp.float32),
        pltpu.VMEM_SHARED((2 * TOT_K,), jnp.int32),
        pltpu.VMEM_SHARED((2 * TOT_K,), jnp.int32),
        pltpu.VMEM_SHARED((NSB * 16,), jnp.int32),
        pltpu.VMEM_SHARED((NSB * 16,), jnp.float32),
    ],
)


def _body_c(scores_ref, gidx_ref, tcls_ref, regv_ref, out_ref, rows_ref, sem):
    copies = []
    for r in range(TOT_K):
        cp = pltpu.make_async_copy(
            scores_ref.at[pl.ds(gidx_ref[r], 1), :],
            rows_ref.at[pl.ds(r, 1), :],
            sem,
        )
        cp.start()
        copies.append(cp)
    for cp in copies:
        cp.wait()

    rows = rows_ref[...]
    m = jnp.max(rows, axis=1, keepdims=True)
    lse = m + jnp.log(jnp.sum(jnp.exp(rows - m), axis=1, keepdims=True))
    t = tcls_ref[...]
    onehot = jax.lax.broadcasted_iota(jnp.int32, (TOT_K, C), 1) == t
    logit_t = jnp.sum(jnp.where(onehot, rows, 0.0), axis=1, keepdims=True)
    cls_t = jnp.sum(logit_t - lse)
    reg = regv_ref[0]
    i2 = jax.lax.broadcasted_iota(jnp.int32, (8, 128), 1)
    r2 = jax.lax.broadcasted_iota(jnp.int32, (8, 128), 0)
    val = jnp.where(i2 == 0, -cls_t * (1.0 / TOT_K), reg)
    out_ref[...] = jnp.where((r2 == 0) & (i2 < 2), val, 0.0)


_call_c = pl.pallas_call(
    _body_c,
    out_shape=jax.ShapeDtypeStruct((8, 128), jnp.float32),
    in_specs=[
        pl.BlockSpec(memory_space=pltpu.HBM),
        pl.BlockSpec(memory_space=pltpu.SMEM),
        pl.BlockSpec(memory_space=pltpu.VMEM),
        pl.BlockSpec(memory_space=pltpu.SMEM),
    ],
    out_specs=pl.BlockSpec(memory_space=pltpu.VMEM),
    scratch_shapes=[
        pltpu.VMEM((TOT_K, C), jnp.float32),
        pltpu.SemaphoreType.DMA,
    ],
)


def kernel(rpn_proposals_bboxes, roi_score, roi_bboxes_txtytwth, gt_bboxes, gt_labels):
    rois_p = jnp.pad(rpn_proposals_bboxes, ((0, NPAD - N), (0, 0))).reshape(-1)
    txty_p = jnp.pad(roi_bboxes_txtytwth, ((0, NPAD - N), (0, 0))).reshape(-1)
    gt_f = gt_bboxes.reshape(-1)
    gtl = gt_labels.astype(jnp.int32)
    v, hist = _call_a(rois_p, gt_f)
    gidx, tcls, regv = _call_b(rois_p, txty_p, gt_f, gtl, v, hist)
    out = _call_c(roi_score, gidx, tcls.reshape(TOT_K, 1), regv)
    return out[0, 0], out[0, 1]


# speedup vs baseline: 1.7704x; 1.1362x over previous
import jax
import jax.numpy as jnp
from jax import lax
from jax.experimental import pallas as pl
from jax.experimental.pallas import tpu as pltpu
from jax.experimental.pallas import tpu_sc as plsc

N = 20000
C = 81
G = 64
NSA = 32
NTA = 640
GRPSA = NTA // 16
NSB = 16
NTB = 1280
GRPSB = NTB // 16
POS_K = 32
NEG_K = 96
TOT_K = 128
REG_W = 2.0

_MESH_A = plsc.VectorSubcoreMesh(
    core_axis_name="c", subcore_axis_name="s", num_cores=2, num_subcores=16
)
_MESH_B = plsc.VectorSubcoreMesh(
    core_axis_name="c", subcore_axis_name="s", num_cores=1, num_subcores=16
)
_SC_PARAMS = pltpu.CompilerParams(needs_layout_passes=False)


def _ln(x):
    bits = lax.bitcast_convert_type(x, jnp.int32)
    e = jnp.right_shift(bits, 23) & 0xFF
    m = lax.bitcast_convert_type((bits & 0x7FFFFF) | 0x3F800000, jnp.float32)
    big = m > 1.4142135623730951
    m = jnp.where(big, m * 0.5, m)
    ef = (e - 127 + big.astype(jnp.int32)).astype(jnp.float32)
    s = (m - 1.0) / (m + 1.0)
    s2 = s * s
    p = jnp.float32(1.0 / 9.0)
    p = p * s2 + jnp.float32(1.0 / 7.0)
    p = p * s2 + jnp.float32(0.2)
    p = p * s2 + jnp.float32(1.0 / 3.0)
    p = p * s2 + jnp.float32(1.0)
    lnm = 2.0 * s * p
    return ef * jnp.float32(0.6931471805599453) + lnm


def _body_a(rois_hbm, gt_hbm, v_hbm, hist_hbm, rois_l, gt_l, v_l, hist_l):
    wid = lax.axis_index("s") * 2 + lax.axis_index("c")
    own_lo = wid * NTA
    dbase = jnp.minimum(own_lo, N - NTA)
    iota = lax.iota(jnp.int32, 16)
    zc = jnp.zeros((16,), jnp.int32)

    pltpu.sync_copy(rois_hbm.at[pl.ds(dbase * 4, NTA * 4)], rois_l)
    pltpu.sync_copy(gt_hbm, gt_l)

    for q in range(4):
        hist_l[pl.ds(q * 16, 16)] = zc

    def group_body(g, _):
        ridx = (g * 16 + iota) * 4
        ax0 = plsc.load_gather(rois_l, [ridx])
        ay0 = plsc.load_gather(rois_l, [ridx + 1])
        ax1 = plsc.load_gather(rois_l, [ridx + 2])
        ay1 = plsc.load_gather(rois_l, [ridx + 3])
        area_a = (ax1 - ax0) * (ay1 - ay0)

        def gt_body(jj, carry):
            best, bidx = carry
            for u in range(4):
                j = jj * 4 + u
                bx0 = plsc.load_gather(gt_l, [zc + j * 4])
                by0 = plsc.load_gather(gt_l, [zc + (j * 4 + 1)])
                bx1 = plsc.load_gather(gt_l, [zc + (j * 4 + 2)])
                by1 = plsc.load_gather(gt_l, [zc + (j * 4 + 3)])
                ab = (bx1 - bx0) * (by1 - by0)
                w = jnp.maximum(jnp.minimum(ax1, bx1) - jnp.maximum(ax0, bx0), 0.0)
                h = jnp.maximum(jnp.minimum(ay1, by1) - jnp.maximum(ay0, by0), 0.0)
                inter = w * h
                iou = inter / (area_a + ab - inter + 1e-8)
                upd = iou > best
                best = jnp.where(upd, iou, best)
                bidx = jnp.where(upd, j, bidx)
            return best, bidx

        best, bidx = lax.fori_loop(
            0, 16, gt_body, (jnp.full((16,), -1.0, jnp.float32), zc)
        )
        v_l[pl.ds(g * 16, 16)] = bidx
        gi = dbase + g * 16 + iota
        valid = jnp.logical_and(gi >= own_lo, gi < N)
        plsc.addupdate_scatter(hist_l, [bidx], zc + 1, mask=valid)
        return 0

    lax.fori_loop(0, GRPSA, group_body, 0)
    pltpu.sync_copy(v_l, v_hbm.at[pl.ds(dbase, NTA)])
    pltpu.sync_copy(hist_l, hist_hbm.at[pl.ds(wid * G, G)])


_call_a = pl.kernel(
    _body_a,
    out_type=(
        jax.ShapeDtypeStruct((N,), jnp.int32),
        jax.ShapeDtypeStruct((NSA * G,), jnp.int32),
    ),
    mesh=_MESH_A,
    compiler_params=_SC_PARAMS,
    scratch_types=[
        pltpu.VMEM((NTA * 4,), jnp.float32),
        pltpu.VMEM((G * 4,), jnp.float32),
        pltpu.VMEM((NTA,), jnp.int32),
        pltpu.VMEM((G,), jnp.int32),
    ],
)


def _body_b(rois_hbm, txty_hbm, gt_hbm, gtl_hbm, v_hbm, hist_hbm,
            gidx_hbm, tcls_hbm, regv_hbm,
            rois_l, txty_l, gt_l, gtl_l, v_l, histall_l, gsum_l, cdf_l,
            sel_l, gidx_l, tcls_l, pos_l, cntst_l, cntall_l, part_l,
            partall_l, out_l,
            sh_gidx, sh_tcls, sh_cnt, sh_part):
    wid = lax.axis_index("s")
    own_lo = wid * NTB
    dbase = jnp.minimum(own_lo, N - NTB)
    iota = lax.iota(jnp.int32, 16)
    zc = jnp.zeros((16,), jnp.int32)
    zf = jnp.zeros((16,), jnp.float32)

    pltpu.sync_copy(rois_hbm.at[pl.ds(dbase * 4, NTB * 4)], rois_l)
    pltpu.sync_copy(txty_hbm.at[pl.ds(dbase * 4, NTB * 4)], txty_l)
    pltpu.sync_copy(gt_hbm, gt_l)
    pltpu.sync_copy(gtl_hbm, gtl_l)
    pltpu.sync_copy(v_hbm.at[pl.ds(dbase, NTB)], v_l)
    pltpu.sync_copy(hist_hbm, histall_l)

    gq = []
    for q in range(4):
        acc = zc
        for w in range(NSA):
            acc = acc + histall_l[pl.ds(w * G + q * 16, 16)]
        gsum_l[pl.ds(q * 16, 16)] = acc
        gq.append(acc)

    cq = []
    tot = jnp.int32(0)
    for q in range(4):
        cc = plsc.cumsum(gq[q]) + tot
        tot = tot + jnp.sum(gq[q])
        cdf_l[pl.ds(q * 16, 16)] = cc
        cq.append(cc)

    npos = zc
    nneg = zc
    for q in range(4):
        cprev = cq[q] - gq[q]
        npos = npos + plsc.all_reduce_population_count(cprev <= N - POS_K)
        nneg = nneg + plsc.all_reduce_population_count(cq[q] < NEG_K)
    tpos = jnp.max(npos) - 1
    tneg = jnp.max(nneg)
    cpos = jnp.max(plsc.load_gather(cdf_l, [zc + tpos]))
    rpos = POS_K - (N - cpos)
    cneg = jnp.max(plsc.load_gather(cdf_l, [zc + tneg]))
    gneg = jnp.max(plsc.load_gather(gsum_l, [zc + tneg]))
    rneg = NEG_K - (cneg - gneg)

    hp0 = plsc.load_gather(histall_l, [iota * G + tpos])
    hp1 = plsc.load_gather(histall_l, [(iota + 16) * G + tpos])
    hn0 = plsc.load_gather(histall_l, [iota * G + tneg])
    hn1 = plsc.load_gather(histall_l, [(iota + 16) * G + tneg])
    a2 = wid * 2
    base_pos = (jnp.sum(jnp.where(iota < a2, hp0, 0))
                + jnp.sum(jnp.where(iota + 16 < a2, hp1, 0)))
    base_neg = (jnp.sum(jnp.where(iota < a2, hn0, 0))
                + jnp.sum(jnp.where(iota + 16 < a2, hn1, 0)))

    for q in range(8):
        sel_l[pl.ds(q * 16, 16)] = zc

    def sel_body(g, carry):
        cntv, tpv, tnv = carry
        v = v_l[pl.ds(g * 16, 16)]
        gi = dbase + g * 16 + iota
        valid = jnp.logical_and(gi >= own_lo, gi < N)
        hi = jnp.logical_and(v > tpos, valid)
        mtp = jnp.logical_and(v == tpos, valid)
        rkp = tpv + plsc.cumsum(mtp.astype(jnp.int32)) - 1 + base_pos
        ptie = jnp.logical_and(mtp, rkp < rpos)
        lo = jnp.logical_and(v < tneg, valid)
        mtn = jnp.logical_and(v == tneg, valid)
        rkn = tnv + plsc.cumsum(mtn.astype(jnp.int32)) - 1 + base_neg
        ntie = jnp.logical_and(mtn, rkn < rneg)
        sel = jnp.logical_or(jnp.logical_or(hi, ptie), jnp.logical_or(lo, ntie))
        pos = cntv + plsc.cumsum(sel.astype(jnp.int32)) - 1
        plsc.store_scatter(sel_l, [pos], g * 16 + iota, mask=sel)
        cntv = cntv + plsc.all_reduce_population_count(sel)
        tpv = tpv + plsc.all_reduce_population_count(mtp)
        tnv = tnv + plsc.all_reduce_population_count(mtn)
        return cntv, tpv, tnv

    cntv, _, _ = lax.fori_loop(0, GRPSB, sel_body, (zc, zc, zc))
    cnt_s = jnp.max(cntv)

    ngrp = jnp.right_shift(cnt_s + 15, 4)

    def loss_body(q, reg_acc):
        rvec = q * 16 + iota
        actf = (rvec < cnt_s).astype(jnp.float32)
        lidx = plsc.load_gather(sel_l, [rvec])
        vr = plsc.load_gather(v_l, [lidx])
        lab = (vr >= 1).astype(jnp.int32)
        labf = lab.astype(jnp.float32)
        gl = plsc.load_gather(gtl_l, [vr])
        tcls_l[pl.ds(q * 16, 16)] = jnp.clip(gl * lab, 0, C - 1)
        gidx_l[pl.ds(q * 16, 16)] = lidx + dbase

        l4 = lidx * 4
        v4 = vr * 4
        ax0 = plsc.load_gather(rois_l, [l4])
        ay0 = plsc.load_gather(rois_l, [l4 + 1])
        ax1 = plsc.load_gather(rois_l, [l4 + 2])
        ay1 = plsc.load_gather(rois_l, [l4 + 3])
        p0 = plsc.load_gather(txty_l, [l4])
        p1 = plsc.load_gather(txty_l, [l4 + 1])
        p2 = plsc.load_gather(txty_l, [l4 + 2])
        p3 = plsc.load_gather(txty_l, [l4 + 3])
        gx0 = plsc.load_gather(gt_l, [v4])
        gy0 = plsc.load_gather(gt_l, [v4 + 1])
        gx1 = plsc.load_gather(gt_l, [v4 + 2])
        gy1 = plsc.load_gather(gt_l, [v4 + 3])
        aw = ax1 - ax0
        ah = ay1 - ay0
        axc = ax0 + 0.5 * aw
        ayc = ay0 + 0.5 * ah
        gw = gx1 - gx0
        gh = gy1 - gy0
        gxc = gx0 + 0.5 * gw
        gyc = gy0 + 0.5 * gh
        awm = jnp.maximum(aw, 1e-8)
        ahm = jnp.maximum(ah, 1e-8)
        tx = (gxc - axc) / awm
        ty = (gyc - ayc) / ahm
        tw = _ln(jnp.maximum(gw, 1e-8) / awm)
        th = _ln(jnp.maximum(gh, 1e-8) / ahm)
        sl1 = zf
        for d in (labf * (p0 - tx), labf * (p1 - ty),
                  labf * (p2 - tw), labf * (p3 - th)):
            ad = jnp.abs(d)
            sl1 = sl1 + jnp.where(ad < 1.0, 0.5 * ad * ad, ad - 0.5)
        return reg_acc + sl1 * actf

    reg_acc = lax.fori_loop(0, ngrp, loss_body, zf)

    cntst_l[...] = cntv
    pltpu.sync_copy(cntst_l, sh_cnt.at[pl.ds(wid * 16, 16)])
    part_l[...] = reg_acc
    pltpu.sync_copy(part_l, sh_part.at[pl.ds(wid * 16, 16)])
    plsc.subcore_barrier()
    pltpu.sync_copy(sh_cnt, cntall_l)
    cnts = plsc.load_gather(cntall_l, [iota * 16])
    offset = jnp.sum(jnp.where(iota < wid, cnts, 0))

    def pos_body(q, _):
        rr = q * 16 + iota
        pos_l[pl.ds(q * 16, 16)] = jnp.where(rr < cnt_s, offset + rr,
                                             TOT_K + rr)
        return 0

    lax.fori_loop(0, 8, pos_body, 0)
    pltpu.sync_copy(gidx_l, sh_gidx.at[pos_l])
    pltpu.sync_copy(tcls_l, sh_tcls.at[pos_l])
    plsc.subcore_barrier()

    @pl.when(wid == 0)
    def _():
        pltpu.sync_copy(sh_gidx.at[pl.ds(0, TOT_K)], gidx_hbm)
        pltpu.sync_copy(sh_tcls.at[pl.ds(0, TOT_K)], tcls_hbm)
        pltpu.sync_copy(sh_part, partall_l)
        rs = zf
        for w in range(NSB):
            rs = rs + partall_l[pl.ds(w * 16, 16)]
        reg_t = jnp.sum(rs)
        out_l[...] = jnp.where(iota == 0, (REG_W / TOT_K) * reg_t, 0.0)
        pltpu.sync_copy(out_l, regv_hbm)


_call_b = pl.kernel(
    _body_b,
    out_type=(
        jax.ShapeDtypeStruct((TOT_K,), jnp.int32),
        jax.ShapeDtypeStruct((TOT_K,), jnp.int32),
        jax.ShapeDtypeStruct((16,), jnp.float32),
    ),
    mesh=_MESH_B,
    compiler_params=_SC_PARAMS,
    scratch_types=[
        pltpu.VMEM((NTB * 4,), jnp.float32),
        pltpu.VMEM((NTB * 4,), jnp.float32),
        pltpu.VMEM((G * 4,), jnp.float32),
        pltpu.VMEM((G,), jnp.int32),
        pltpu.VMEM((NTB,), jnp.int32),
        pltpu.VMEM((NSA * G,), jnp.int32),
        pltpu.VMEM((G,), jnp.int32),
        pltpu.VMEM((G,), jnp.int32),
        pltpu.VMEM((TOT_K,), jnp.int32),
        pltpu.VMEM((TOT_K,), jnp.int32),
        pltpu.VMEM((TOT_K,), jnp.int32),
        pltpu.VMEM((TOT_K,), jnp.int32),
        pltpu.VMEM((16,), jnp.int32),
        pltpu.VMEM((NSB * 16,), jnp.int32),
        pltpu.VMEM((16,), jnp.float32),
        pltpu.VMEM((NSB * 16,), jnp.float32),
        pltpu.VMEM((16,), jnp.float32),
        pltpu.VMEM_SHARED((2 * TOT_K,), jnp.int32),
        pltpu.VMEM_SHARED((2 * TOT_K,), jnp.int32),
        pltpu.VMEM_SHARED((NSB * 16,), jnp.int32),
        pltpu.VMEM_SHARED((NSB * 16,), jnp.float32),
    ],
)


def _body_c(scores_ref, gidx_s, gidx_v, tcls_v, regv_ref, out_ref,
            rows8_ref, sem):
    copies = []
    for r in range(TOT_K):
        tb = pl.multiple_of((gidx_s[r] >> 3) * 8, 8)
        cp = pltpu.make_async_copy(
            scores_ref.at[pl.ds(tb, 8), :],
            rows8_ref.at[pl.ds(r * 8, 8), :],
            sem,
        )
        cp.start()
        copies.append(cp)
    for cp in copies:
        cp.wait()

    rem_col = jnp.transpose((gidx_v[...] & 7).reshape(1, TOT_K))
    t_col = jnp.transpose(tcls_v[...].reshape(1, TOT_K))
    ci = jax.lax.broadcasted_iota(jnp.int32, (TOT_K, TOT_K * 8), 1)
    ri = jax.lax.broadcasted_iota(jnp.int32, (TOT_K, TOT_K * 8), 0)
    sel = (ci == ri * 8 + rem_col).astype(jnp.float32)
    rows = jax.lax.dot_general(
        sel, rows8_ref[...], (((1,), (0,)), ((), ())),
        preferred_element_type=jnp.float32)
    m = jnp.max(rows, axis=1, keepdims=True)
    lse = m + jnp.log(jnp.sum(jnp.exp(rows - m), axis=1, keepdims=True))
    onehot = jax.lax.broadcasted_iota(jnp.int32, (TOT_K, C), 1) == t_col
    logit_t = jnp.sum(jnp.where(onehot, rows, 0.0), axis=1, keepdims=True)
    cls_t = jnp.sum(logit_t - lse)
    reg = regv_ref[0]
    i2 = jax.lax.broadcasted_iota(jnp.int32, (8, 128), 1)
    r2 = jax.lax.broadcasted_iota(jnp.int32, (8, 128), 0)
    val = jnp.where(i2 == 0, -cls_t * (1.0 / TOT_K), reg)
    out_ref[...] = jnp.where((r2 == 0) & (i2 < 2), val, 0.0)


_call_c = pl.pallas_call(
    _body_c,
    out_shape=jax.ShapeDtypeStruct((8, 128), jnp.float32),
    in_specs=[
        pl.BlockSpec(memory_space=pltpu.HBM),
        pl.BlockSpec(memory_space=pltpu.SMEM),
        pl.BlockSpec(memory_space=pltpu.VMEM),
        pl.BlockSpec(memory_space=pltpu.VMEM),
        pl.BlockSpec(memory_space=pltpu.SMEM),
    ],
    out_specs=pl.BlockSpec(memory_space=pltpu.VMEM),
    scratch_shapes=[
        pltpu.VMEM((TOT_K * 8, C), jnp.float32),
        pltpu.SemaphoreType.DMA,
    ],
)


def kernel(rpn_proposals_bboxes, roi_score, roi_bboxes_txtytwth, gt_bboxes, gt_labels):
    rois_f = rpn_proposals_bboxes.reshape(-1)
    txty_f = roi_bboxes_txtytwth.reshape(-1)
    gt_f = gt_bboxes.reshape(-1)
    gtl = gt_labels.astype(jnp.int32)
    v, hist = _call_a(rois_f, gt_f)
    gidx, tcls, regv = _call_b(rois_f, txty_f, gt_f, gtl, v, hist)
    out = _call_c(roi_score, gidx, gidx, tcls, regv)
    return out[0, 0], out[0, 1]


# speedup vs baseline: 1.7750x; 1.0026x over previous
import jax
import jax.numpy as jnp
from jax import lax
from jax.experimental import pallas as pl
from jax.experimental.pallas import tpu as pltpu
from jax.experimental.pallas import tpu_sc as plsc

N = 20000
C = 81
G = 64
NSA = 32
NTA = 640
GRPSA = NTA // 16
NSB = 16
NTB = 1280
GRPSB = NTB // 16
POS_K = 32
NEG_K = 96
TOT_K = 128
REG_W = 2.0

_MESH_A = plsc.VectorSubcoreMesh(
    core_axis_name="c", subcore_axis_name="s", num_cores=2, num_subcores=16
)
_MESH_B = plsc.VectorSubcoreMesh(
    core_axis_name="c", subcore_axis_name="s", num_cores=1, num_subcores=16
)
_SC_PARAMS = pltpu.CompilerParams(needs_layout_passes=False)


def _ln(x):
    bits = lax.bitcast_convert_type(x, jnp.int32)
    e = jnp.right_shift(bits, 23) & 0xFF
    m = lax.bitcast_convert_type((bits & 0x7FFFFF) | 0x3F800000, jnp.float32)
    big = m > 1.4142135623730951
    m = jnp.where(big, m * 0.5, m)
    ef = (e - 127 + big.astype(jnp.int32)).astype(jnp.float32)
    s = (m - 1.0) / (m + 1.0)
    s2 = s * s
    p = jnp.float32(1.0 / 9.0)
    p = p * s2 + jnp.float32(1.0 / 7.0)
    p = p * s2 + jnp.float32(0.2)
    p = p * s2 + jnp.float32(1.0 / 3.0)
    p = p * s2 + jnp.float32(1.0)
    lnm = 2.0 * s * p
    return ef * jnp.float32(0.6931471805599453) + lnm


def _body_a(rois_hbm, gt_hbm, v_hbm, hist_hbm, rois_l, gt_l, areab_l,
            v_l, hist_l):
    wid = lax.axis_index("s") * 2 + lax.axis_index("c")
    own_lo = wid * NTA
    dbase = jnp.minimum(own_lo, N - NTA)
    iota = lax.iota(jnp.int32, 16)
    zc = jnp.zeros((16,), jnp.int32)

    pltpu.sync_copy(rois_hbm.at[pl.ds(dbase * 4, NTA * 4)], rois_l)
    pltpu.sync_copy(gt_hbm, gt_l)

    for q in range(4):
        hist_l[pl.ds(q * 16, 16)] = zc
        gidx16 = (q * 16 + iota) * 4
        bx0 = plsc.load_gather(gt_l, [gidx16])
        by0 = plsc.load_gather(gt_l, [gidx16 + 1])
        bx1 = plsc.load_gather(gt_l, [gidx16 + 2])
        by1 = plsc.load_gather(gt_l, [gidx16 + 3])
        areab_l[pl.ds(q * 16, 16)] = (bx1 - bx0) * (by1 - by0)

    def group_body(g, _):
        ridx = (g * 16 + iota) * 4
        ax0 = plsc.load_gather(rois_l, [ridx])
        ay0 = plsc.load_gather(rois_l, [ridx + 1])
        ax1 = plsc.load_gather(rois_l, [ridx + 2])
        ay1 = plsc.load_gather(rois_l, [ridx + 3])
        area_a = (ax1 - ax0) * (ay1 - ay0)

        def gt_body(jj, carry):
            best, bidx = carry
            for u in range(8):
                j = jj * 8 + u
                bx0 = plsc.load_gather(gt_l, [zc + j * 4])
                by0 = plsc.load_gather(gt_l, [zc + (j * 4 + 1)])
                bx1 = plsc.load_gather(gt_l, [zc + (j * 4 + 2)])
                by1 = plsc.load_gather(gt_l, [zc + (j * 4 + 3)])
                ab = plsc.load_gather(areab_l, [zc + j])
                w = jnp.maximum(jnp.minimum(ax1, bx1) - jnp.maximum(ax0, bx0), 0.0)
                h = jnp.maximum(jnp.minimum(ay1, by1) - jnp.maximum(ay0, by0), 0.0)
                inter = w * h
                iou = inter / (area_a + ab - inter + 1e-8)
                upd = iou > best
                best = jnp.where(upd, iou, best)
                bidx = jnp.where(upd, j, bidx)
            return best, bidx

        best, bidx = lax.fori_loop(
            0, 8, gt_body, (jnp.full((16,), -1.0, jnp.float32), zc)
        )
        v_l[pl.ds(g * 16, 16)] = bidx
        gi = dbase + g * 16 + iota
        valid = jnp.logical_and(gi >= own_lo, gi < N)
        plsc.addupdate_scatter(hist_l, [bidx], zc + 1, mask=valid)
        return 0

    lax.fori_loop(0, GRPSA, group_body, 0)
    pltpu.sync_copy(v_l, v_hbm.at[pl.ds(dbase, NTA)])
    pltpu.sync_copy(hist_l, hist_hbm.at[pl.ds(wid * G, G)])


_call_a = pl.kernel(
    _body_a,
    out_type=(
        jax.ShapeDtypeStruct((N,), jnp.int32),
        jax.ShapeDtypeStruct((NSA * G,), jnp.int32),
    ),
    mesh=_MESH_A,
    compiler_params=_SC_PARAMS,
    scratch_types=[
        pltpu.VMEM((NTA * 4,), jnp.float32),
        pltpu.VMEM((G * 4,), jnp.float32),
        pltpu.VMEM((G,), jnp.float32),
        pltpu.VMEM((NTA,), jnp.int32),
        pltpu.VMEM((G,), jnp.int32),
    ],
)


def _body_b(rois_hbm, txty_hbm, gt_hbm, gtl_hbm, v_hbm, hist_hbm,
            gidx_hbm, tcls_hbm, regv_hbm,
            rois_l, txty_l, gt_l, gtl_l, v_l, histall_l, gsum_l, cdf_l,
            sel_l, gidx_l, tcls_l, pos_l, cntst_l, cntall_l, part_l,
            partall_l, out_l,
            sh_gidx, sh_tcls, sh_cnt, sh_part):
    wid = lax.axis_index("s")
    own_lo = wid * NTB
    dbase = jnp.minimum(own_lo, N - NTB)
    iota = lax.iota(jnp.int32, 16)
    zc = jnp.zeros((16,), jnp.int32)
    zf = jnp.zeros((16,), jnp.float32)

    pltpu.sync_copy(rois_hbm.at[pl.ds(dbase * 4, NTB * 4)], rois_l)
    pltpu.sync_copy(txty_hbm.at[pl.ds(dbase * 4, NTB * 4)], txty_l)
    pltpu.sync_copy(gt_hbm, gt_l)
    pltpu.sync_copy(gtl_hbm, gtl_l)
    pltpu.sync_copy(v_hbm.at[pl.ds(dbase, NTB)], v_l)
    pltpu.sync_copy(hist_hbm, histall_l)

    gq = []
    for q in range(4):
        acc = zc
        for w in range(NSA):
            acc = acc + histall_l[pl.ds(w * G + q * 16, 16)]
        gsum_l[pl.ds(q * 16, 16)] = acc
        gq.append(acc)

    cq = []
    tot = jnp.int32(0)
    for q in range(4):
        cc = plsc.cumsum(gq[q]) + tot
        tot = tot + jnp.sum(gq[q])
        cdf_l[pl.ds(q * 16, 16)] = cc
        cq.append(cc)

    npos = zc
    nneg = zc
    for q in range(4):
        cprev = cq[q] - gq[q]
        npos = npos + plsc.all_reduce_population_count(cprev <= N - POS_K)
        nneg = nneg + plsc.all_reduce_population_count(cq[q] < NEG_K)
    tpos = jnp.max(npos) - 1
    tneg = jnp.max(nneg)
    cpos = jnp.max(plsc.load_gather(cdf_l, [zc + tpos]))
    rpos = POS_K - (N - cpos)
    cneg = jnp.max(plsc.load_gather(cdf_l, [zc + tneg]))
    gneg = jnp.max(plsc.load_gather(gsum_l, [zc + tneg]))
    rneg = NEG_K - (cneg - gneg)

    hp0 = plsc.load_gather(histall_l, [iota * G + tpos])
    hp1 = plsc.load_gather(histall_l, [(iota + 16) * G + tpos])
    hn0 = plsc.load_gather(histall_l, [iota * G + tneg])
    hn1 = plsc.load_gather(histall_l, [(iota + 16) * G + tneg])
    a2 = wid * 2
    base_pos = (jnp.sum(jnp.where(iota < a2, hp0, 0))
                + jnp.sum(jnp.where(iota + 16 < a2, hp1, 0)))
    base_neg = (jnp.sum(jnp.where(iota < a2, hn0, 0))
                + jnp.sum(jnp.where(iota + 16 < a2, hn1, 0)))

    for q in range(8):
        sel_l[pl.ds(q * 16, 16)] = zc

    def sel_body(g, carry):
        cntv, tpv, tnv = carry
        v = v_l[pl.ds(g * 16, 16)]
        gi = dbase + g * 16 + iota
        valid = jnp.logical_and(gi >= own_lo, gi < N)
        hi = jnp.logical_and(v > tpos, valid)
        mtp = jnp.logical_and(v == tpos, valid)
        rkp = tpv + plsc.cumsum(mtp.astype(jnp.int32)) - 1 + base_pos
        ptie = jnp.logical_and(mtp, rkp < rpos)
        lo = jnp.logical_and(v < tneg, valid)
        mtn = jnp.logical_and(v == tneg, valid)
        rkn = tnv + plsc.cumsum(mtn.astype(jnp.int32)) - 1 + base_neg
        ntie = jnp.logical_and(mtn, rkn < rneg)
        sel = jnp.logical_or(jnp.logical_or(hi, ptie), jnp.logical_or(lo, ntie))
        pos = cntv + plsc.cumsum(sel.astype(jnp.int32)) - 1
        plsc.store_scatter(sel_l, [pos], g * 16 + iota, mask=sel)
        cntv = cntv + plsc.all_reduce_population_count(sel)
        tpv = tpv + plsc.all_reduce_population_count(mtp)
        tnv = tnv + plsc.all_reduce_population_count(mtn)
        return cntv, tpv, tnv

    cntv, _, _ = lax.fori_loop(0, GRPSB, sel_body, (zc, zc, zc))
    cnt_s = jnp.max(cntv)

    ngrp = jnp.right_shift(cnt_s + 15, 4)

    def loss_body(q, reg_acc):
        rvec = q * 16 + iota
        actf = (rvec < cnt_s).astype(jnp.float32)
        lidx = plsc.load_gather(sel_l, [rvec])
        vr = plsc.load_gather(v_l, [lidx])
        lab = (vr >= 1).astype(jnp.int32)
        labf = lab.astype(jnp.float32)
        gl = plsc.load_gather(gtl_l, [vr])
        tcls_l[pl.ds(q * 16, 16)] = jnp.clip(gl * lab, 0, C - 1)
        gidx_l[pl.ds(q * 16, 16)] = lidx + dbase

        l4 = lidx * 4
        v4 = vr * 4
        ax0 = plsc.load_gather(rois_l, [l4])
        ay0 = plsc.load_gather(rois_l, [l4 + 1])
        ax1 = plsc.load_gather(rois_l, [l4 + 2])
        ay1 = plsc.load_gather(rois_l, [l4 + 3])
        p0 = plsc.load_gather(txty_l, [l4])
        p1 = plsc.load_gather(txty_l, [l4 + 1])
        p2 = plsc.load_gather(txty_l, [l4 + 2])
        p3 = plsc.load_gather(txty_l, [l4 + 3])
        gx0 = plsc.load_gather(gt_l, [v4])
        gy0 = plsc.load_gather(gt_l, [v4 + 1])
        gx1 = plsc.load_gather(gt_l, [v4 + 2])
        gy1 = plsc.load_gather(gt_l, [v4 + 3])
        aw = ax1 - ax0
        ah = ay1 - ay0
        axc = ax0 + 0.5 * aw
        ayc = ay0 + 0.5 * ah
        gw = gx1 - gx0
        gh = gy1 - gy0
        gxc = gx0 + 0.5 * gw
        gyc = gy0 + 0.5 * gh
        awm = jnp.maximum(aw, 1e-8)
        ahm = jnp.maximum(ah, 1e-8)
        tx = (gxc - axc) / awm
        ty = (gyc - ayc) / ahm
        tw = _ln(jnp.maximum(gw, 1e-8) / awm)
        th = _ln(jnp.maximum(gh, 1e-8) / ahm)
        sl1 = zf
        for d in (labf * (p0 - tx), labf * (p1 - ty),
                  labf * (p2 - tw), labf * (p3 - th)):
            ad = jnp.abs(d)
            sl1 = sl1 + jnp.where(ad < 1.0, 0.5 * ad * ad, ad - 0.5)
        return reg_acc + sl1 * actf

    reg_acc = lax.fori_loop(0, ngrp, loss_body, zf)

    cntst_l[...] = cntv
    pltpu.sync_copy(cntst_l, sh_cnt.at[pl.ds(wid * 16, 16)])
    part_l[...] = reg_acc
    pltpu.sync_copy(part_l, sh_part.at[pl.ds(wid * 16, 16)])
    plsc.subcore_barrier()
    pltpu.sync_copy(sh_cnt, cntall_l)
    cnts = plsc.load_gather(cntall_l, [iota * 16])
    offset = jnp.sum(jnp.where(iota < wid, cnts, 0))

    def pos_body(q, _):
        rr = q * 16 + iota
        pos_l[pl.ds(q * 16, 16)] = jnp.where(rr < cnt_s, offset + rr,
                                             TOT_K + rr)
        return 0

    lax.fori_loop(0, 8, pos_body, 0)
    pltpu.sync_copy(gidx_l, sh_gidx.at[pos_l])
    pltpu.sync_copy(tcls_l, sh_tcls.at[pos_l])
    plsc.subcore_barrier()

    @pl.when(wid == 0)
    def _():
        pltpu.sync_copy(sh_gidx.at[pl.ds(0, TOT_K)], gidx_hbm)
        pltpu.sync_copy(sh_tcls.at[pl.ds(0, TOT_K)], tcls_hbm)
        pltpu.sync_copy(sh_part, partall_l)
        rs = zf
        for w in range(NSB):
            rs = rs + partall_l[pl.ds(w * 16, 16)]
        reg_t = jnp.sum(rs)
        out_l[...] = jnp.where(iota == 0, (REG_W / TOT_K) * reg_t, 0.0)
        pltpu.sync_copy(out_l, regv_hbm)


_call_b = pl.kernel(
    _body_b,
    out_type=(
        jax.ShapeDtypeStruct((TOT_K,), jnp.int32),
        jax.ShapeDtypeStruct((TOT_K,), jnp.int32),
        jax.ShapeDtypeStruct((16,), jnp.float32),
    ),
    mesh=_MESH_B,
    compiler_params=_SC_PARAMS,
    scratch_types=[
        pltpu.VMEM((NTB * 4,), jnp.float32),
        pltpu.VMEM((NTB * 4,), jnp.float32),
        pltpu.VMEM((G * 4,), jnp.float32),
        pltpu.VMEM((G,), jnp.int32),
        pltpu.VMEM((NTB,), jnp.int32),
        pltpu.VMEM((NSA * G,), jnp.int32),
        pltpu.VMEM((G,), jnp.int32),
        pltpu.VMEM((G,), jnp.int32),
        pltpu.VMEM((TOT_K,), jnp.int32),
        pltpu.VMEM((TOT_K,), jnp.int32),
        pltpu.VMEM((TOT_K,), jnp.int32),
        pltpu.VMEM((TOT_K,), jnp.int32),
        pltpu.VMEM((16,), jnp.int32),
        pltpu.VMEM((NSB * 16,), jnp.int32),
        pltpu.VMEM((16,), jnp.float32),
        pltpu.VMEM((NSB * 16,), jnp.float32),
        pltpu.VMEM((16,), jnp.float32),
        pltpu.VMEM_SHARED((2 * TOT_K,), jnp.int32),
        pltpu.VMEM_SHARED((2 * TOT_K,), jnp.int32),
        pltpu.VMEM_SHARED((NSB * 16,), jnp.int32),
        pltpu.VMEM_SHARED((NSB * 16,), jnp.float32),
    ],
)


def _body_c(scores_ref, gidx_s, gidx_v, tcls_v, regv_ref, out_ref,
            rows8_ref, sem):
    copies = []
    for r in range(TOT_K):
        tb = pl.multiple_of((gidx_s[r] >> 3) * 8, 8)
        cp = pltpu.make_async_copy(
            scores_ref.at[pl.ds(tb, 8), :],
            rows8_ref.at[pl.ds(r * 8, 8), :],
            sem,
        )
        cp.start()
        copies.append(cp)
    for cp in copies:
        cp.wait()

    rem_col = jnp.transpose((gidx_v[...] & 7).reshape(1, TOT_K))
    t_col = jnp.transpose(tcls_v[...].reshape(1, TOT_K))
    ci = jax.lax.broadcasted_iota(jnp.int32, (TOT_K, TOT_K * 8), 1)
    ri = jax.lax.broadcasted_iota(jnp.int32, (TOT_K, TOT_K * 8), 0)
    sel = (ci == ri * 8 + rem_col).astype(jnp.float32)
    rows = jax.lax.dot_general(
        sel, rows8_ref[...], (((1,), (0,)), ((), ())),
        preferred_element_type=jnp.float32)
    m = jnp.max(rows, axis=1, keepdims=True)
    lse = m + jnp.log(jnp.sum(jnp.exp(rows - m), axis=1, keepdims=True))
    onehot = jax.lax.broadcasted_iota(jnp.int32, (TOT_K, C), 1) == t_col
    logit_t = jnp.sum(jnp.where(onehot, rows, 0.0), axis=1, keepdims=True)
    cls_t = jnp.sum(logit_t - lse)
    reg = regv_ref[0]
    i2 = jax.lax.broadcasted_iota(jnp.int32, (8, 128), 1)
    r2 = jax.lax.broadcasted_iota(jnp.int32, (8, 128), 0)
    val = jnp.where(i2 == 0, -cls_t * (1.0 / TOT_K), reg)
    out_ref[...] = jnp.where((r2 == 0) & (i2 < 2), val, 0.0)


_call_c = pl.pallas_call(
    _body_c,
    out_shape=jax.ShapeDtypeStruct((8, 128), jnp.float32),
    in_specs=[
        pl.BlockSpec(memory_space=pltpu.HBM),
        pl.BlockSpec(memory_space=pltpu.SMEM),
        pl.BlockSpec(memory_space=pltpu.VMEM),
        pl.BlockSpec(memory_space=pltpu.VMEM),
        pl.BlockSpec(memory_space=pltpu.SMEM),
    ],
    out_specs=pl.BlockSpec(memory_space=pltpu.VMEM),
    scratch_shapes=[
        pltpu.VMEM((TOT_K * 8, C), jnp.float32),
        pltpu.SemaphoreType.DMA,
    ],
)


def kernel(rpn_proposals_bboxes, roi_score, roi_bboxes_txtytwth, gt_bboxes, gt_labels):
    rois_f = rpn_proposals_bboxes.reshape(-1)
    txty_f = roi_bboxes_txtytwth.reshape(-1)
    gt_f = gt_bboxes.reshape(-1)
    gtl = gt_labels.astype(jnp.int32)
    v, hist = _call_a(rois_f, gt_f)
    gidx, tcls, regv = _call_b(rois_f, txty_f, gt_f, gtl, v, hist)
    out = _call_c(roi_score, gidx, gidx, tcls, regv)
    return out[0, 0], out[0, 1]


# speedup vs baseline: 1.7778x; 1.0016x over previous
import jax
import jax.numpy as jnp
from jax import lax
from jax.experimental import pallas as pl
from jax.experimental.pallas import tpu as pltpu
from jax.experimental.pallas import tpu_sc as plsc

N = 20000
C = 81
G = 64
NSA = 32
NTA = 640
GRPSA = NTA // 16
NSB = 16
NTB = 1280
GRPSB = NTB // 16
POS_K = 32
NEG_K = 96
TOT_K = 128
REG_W = 2.0

_MESH_A = plsc.VectorSubcoreMesh(
    core_axis_name="c", subcore_axis_name="s", num_cores=2, num_subcores=16
)
_MESH_B = plsc.VectorSubcoreMesh(
    core_axis_name="c", subcore_axis_name="s", num_cores=1, num_subcores=16
)
_SC_PARAMS = pltpu.CompilerParams(needs_layout_passes=False)


def _ln(x):
    bits = lax.bitcast_convert_type(x, jnp.int32)
    e = jnp.right_shift(bits, 23) & 0xFF
    m = lax.bitcast_convert_type((bits & 0x7FFFFF) | 0x3F800000, jnp.float32)
    big = m > 1.4142135623730951
    m = jnp.where(big, m * 0.5, m)
    ef = (e - 127 + big.astype(jnp.int32)).astype(jnp.float32)
    s = (m - 1.0) / (m + 1.0)
    s2 = s * s
    p = jnp.float32(1.0 / 9.0)
    p = p * s2 + jnp.float32(1.0 / 7.0)
    p = p * s2 + jnp.float32(0.2)
    p = p * s2 + jnp.float32(1.0 / 3.0)
    p = p * s2 + jnp.float32(1.0)
    lnm = 2.0 * s * p
    return ef * jnp.float32(0.6931471805599453) + lnm


def _body_a(rois_hbm, gt_hbm, v_hbm, hist_hbm, rois_l, gt_l, areab_l,
            v_l, hist_l):
    wid = lax.axis_index("s") * 2 + lax.axis_index("c")
    own_lo = wid * NTA
    dbase = jnp.minimum(own_lo, N - NTA)
    iota = lax.iota(jnp.int32, 16)
    zc = jnp.zeros((16,), jnp.int32)

    pltpu.sync_copy(rois_hbm.at[pl.ds(dbase * 4, NTA * 4)], rois_l)
    pltpu.sync_copy(gt_hbm, gt_l)

    for q in range(4):
        hist_l[pl.ds(q * 16, 16)] = zc
        gidx16 = (q * 16 + iota) * 4
        bx0 = plsc.load_gather(gt_l, [gidx16])
        by0 = plsc.load_gather(gt_l, [gidx16 + 1])
        bx1 = plsc.load_gather(gt_l, [gidx16 + 2])
        by1 = plsc.load_gather(gt_l, [gidx16 + 3])
        areab_l[pl.ds(q * 16, 16)] = (bx1 - bx0) * (by1 - by0)

    def group_body(g, _):
        ridx = (g * 16 + iota) * 4
        ax0 = plsc.load_gather(rois_l, [ridx])
        ay0 = plsc.load_gather(rois_l, [ridx + 1])
        ax1 = plsc.load_gather(rois_l, [ridx + 2])
        ay1 = plsc.load_gather(rois_l, [ridx + 3])
        area_a = (ax1 - ax0) * (ay1 - ay0)

        def one_gt(j, best, bidx):
            bx0 = plsc.load_gather(gt_l, [zc + j * 4])
            by0 = plsc.load_gather(gt_l, [zc + (j * 4 + 1)])
            bx1 = plsc.load_gather(gt_l, [zc + (j * 4 + 2)])
            by1 = plsc.load_gather(gt_l, [zc + (j * 4 + 3)])
            ab = plsc.load_gather(areab_l, [zc + j])
            w = jnp.maximum(jnp.minimum(ax1, bx1) - jnp.maximum(ax0, bx0), 0.0)
            h = jnp.maximum(jnp.minimum(ay1, by1) - jnp.maximum(ay0, by0), 0.0)
            inter = w * h
            iou = inter / (area_a + ab - inter + 1e-8)
            upd = iou > best
            return jnp.where(upd, iou, best), jnp.where(upd, j, bidx)

        def gt_body(jj, carry):
            b1, i1, b2, i2 = carry
            for u in range(4):
                j = jj * 4 + u
                b1, i1 = one_gt(j, b1, i1)
                b2, i2 = one_gt(j + 32, b2, i2)
            return b1, i1, b2, i2

        neg1 = jnp.full((16,), -1.0, jnp.float32)
        b1, i1, b2, i2 = lax.fori_loop(
            0, 8, gt_body, (neg1, zc, neg1, zc)
        )
        upd = b2 > b1
        bidx = jnp.where(upd, i2, i1)
        v_l[pl.ds(g * 16, 16)] = bidx
        gi = dbase + g * 16 + iota
        valid = jnp.logical_and(gi >= own_lo, gi < N)
        plsc.addupdate_scatter(hist_l, [bidx], zc + 1, mask=valid)
        return 0

    lax.fori_loop(0, GRPSA, group_body, 0)
    pltpu.sync_copy(v_l, v_hbm.at[pl.ds(dbase, NTA)])
    pltpu.sync_copy(hist_l, hist_hbm.at[pl.ds(wid * G, G)])


_call_a = pl.kernel(
    _body_a,
    out_type=(
        jax.ShapeDtypeStruct((N,), jnp.int32),
        jax.ShapeDtypeStruct((NSA * G,), jnp.int32),
    ),
    mesh=_MESH_A,
    compiler_params=_SC_PARAMS,
    scratch_types=[
        pltpu.VMEM((NTA * 4,), jnp.float32),
        pltpu.VMEM((G * 4,), jnp.float32),
        pltpu.VMEM((G,), jnp.float32),
        pltpu.VMEM((NTA,), jnp.int32),
        pltpu.VMEM((G,), jnp.int32),
    ],
)


def _body_b(rois_hbm, txty_hbm, gt_hbm, gtl_hbm, v_hbm, hist_hbm,
            gidx_hbm, tcls_hbm, regv_hbm,
            rois_l, txty_l, gt_l, gtl_l, v_l, histall_l, gsum_l, cdf_l,
            sel_l, gidx_l, tcls_l, pos_l, cntst_l, cntall_l, part_l,
            partall_l, out_l,
            sh_gidx, sh_tcls, sh_cnt, sh_part):
    wid = lax.axis_index("s")
    own_lo = wid * NTB
    dbase = jnp.minimum(own_lo, N - NTB)
    iota = lax.iota(jnp.int32, 16)
    zc = jnp.zeros((16,), jnp.int32)
    zf = jnp.zeros((16,), jnp.float32)

    pltpu.sync_copy(rois_hbm.at[pl.ds(dbase * 4, NTB * 4)], rois_l)
    pltpu.sync_copy(txty_hbm.at[pl.ds(dbase * 4, NTB * 4)], txty_l)
    pltpu.sync_copy(gt_hbm, gt_l)
    pltpu.sync_copy(gtl_hbm, gtl_l)
    pltpu.sync_copy(v_hbm.at[pl.ds(dbase, NTB)], v_l)
    pltpu.sync_copy(hist_hbm, histall_l)

    gq = []
    for q in range(4):
        acc = zc
        for w in range(NSA):
            acc = acc + histall_l[pl.ds(w * G + q * 16, 16)]
        gsum_l[pl.ds(q * 16, 16)] = acc
        gq.append(acc)

    cq = []
    tot = jnp.int32(0)
    for q in range(4):
        cc = plsc.cumsum(gq[q]) + tot
        tot = tot + jnp.sum(gq[q])
        cdf_l[pl.ds(q * 16, 16)] = cc
        cq.append(cc)

    npos = zc
    nneg = zc
    for q in range(4):
        cprev = cq[q] - gq[q]
        npos = npos + plsc.all_reduce_population_count(cprev <= N - POS_K)
        nneg = nneg + plsc.all_reduce_population_count(cq[q] < NEG_K)
    tpos = jnp.max(npos) - 1
    tneg = jnp.max(nneg)
    cpos = jnp.max(plsc.load_gather(cdf_l, [zc + tpos]))
    rpos = POS_K - (N - cpos)
    cneg = jnp.max(plsc.load_gather(cdf_l, [zc + tneg]))
    gneg = jnp.max(plsc.load_gather(gsum_l, [zc + tneg]))
    rneg = NEG_K - (cneg - gneg)

    hp0 = plsc.load_gather(histall_l, [iota * G + tpos])
    hp1 = plsc.load_gather(histall_l, [(iota + 16) * G + tpos])
    hn0 = plsc.load_gather(histall_l, [iota * G + tneg])
    hn1 = plsc.load_gather(histall_l, [(iota + 16) * G + tneg])
    a2 = wid * 2
    base_pos = (jnp.sum(jnp.where(iota < a2, hp0, 0))
                + jnp.sum(jnp.where(iota + 16 < a2, hp1, 0)))
    base_neg = (jnp.sum(jnp.where(iota < a2, hn0, 0))
                + jnp.sum(jnp.where(iota + 16 < a2, hn1, 0)))

    for q in range(8):
        sel_l[pl.ds(q * 16, 16)] = zc

    def sel_body(g, carry):
        cntv, tpv, tnv = carry
        v = v_l[pl.ds(g * 16, 16)]
        gi = dbase + g * 16 + iota
        valid = jnp.logical_and(gi >= own_lo, gi < N)
        hi = jnp.logical_and(v > tpos, valid)
        mtp = jnp.logical_and(v == tpos, valid)
        rkp = tpv + plsc.cumsum(mtp.astype(jnp.int32)) - 1 + base_pos
        ptie = jnp.logical_and(mtp, rkp < rpos)
        lo = jnp.logical_and(v < tneg, valid)
        mtn = jnp.logical_and(v == tneg, valid)
        rkn = tnv + plsc.cumsum(mtn.astype(jnp.int32)) - 1 + base_neg
        ntie = jnp.logical_and(mtn, rkn < rneg)
        sel = jnp.logical_or(jnp.logical_or(hi, ptie), jnp.logical_or(lo, ntie))
        pos = cntv + plsc.cumsum(sel.astype(jnp.int32)) - 1
        plsc.store_scatter(sel_l, [pos], g * 16 + iota, mask=sel)
        cntv = cntv + plsc.all_reduce_population_count(sel)
        tpv = tpv + plsc.all_reduce_population_count(mtp)
        tnv = tnv + plsc.all_reduce_population_count(mtn)
        return cntv, tpv, tnv

    cntv, _, _ = lax.fori_loop(0, GRPSB, sel_body, (zc, zc, zc))
    cnt_s = jnp.max(cntv)

    ngrp = jnp.right_shift(cnt_s + 15, 4)

    def loss_body(q, reg_acc):
        rvec = q * 16 + iota
        actf = (rvec < cnt_s).astype(jnp.float32)
        lidx = plsc.load_gather(sel_l, [rvec])
        vr = plsc.load_gather(v_l, [lidx])
        lab = (vr >= 1).astype(jnp.int32)
        labf = lab.astype(jnp.float32)
        gl = plsc.load_gather(gtl_l, [vr])
        tcls_l[pl.ds(q * 16, 16)] = jnp.clip(gl * lab, 0, C - 1)
        gidx_l[pl.ds(q * 16, 16)] = lidx + dbase

        l4 = lidx * 4
        v4 = vr * 4
        ax0 = plsc.load_gather(rois_l, [l4])
        ay0 = plsc.load_gather(rois_l, [l4 + 1])
        ax1 = plsc.load_gather(rois_l, [l4 + 2])
        ay1 = plsc.load_gather(rois_l, [l4 + 3])
        p0 = plsc.load_gather(txty_l, [l4])
        p1 = plsc.load_gather(txty_l, [l4 + 1])
        p2 = plsc.load_gather(txty_l, [l4 + 2])
        p3 = plsc.load_gather(txty_l, [l4 + 3])
        gx0 = plsc.load_gather(gt_l, [v4])
        gy0 = plsc.load_gather(gt_l, [v4 + 1])
        gx1 = plsc.load_gather(gt_l, [v4 + 2])
        gy1 = plsc.load_gather(gt_l, [v4 + 3])
        aw = ax1 - ax0
        ah = ay1 - ay0
        axc = ax0 + 0.5 * aw
        ayc = ay0 + 0.5 * ah
        gw = gx1 - gx0
        gh = gy1 - gy0
        gxc = gx0 + 0.5 * gw
        gyc = gy0 + 0.5 * gh
        awm = jnp.maximum(aw, 1e-8)
        ahm = jnp.maximum(ah, 1e-8)
        tx = (gxc - axc) / awm
        ty = (gyc - ayc) / ahm
        tw = _ln(jnp.maximum(gw, 1e-8) / awm)
        th = _ln(jnp.maximum(gh, 1e-8) / ahm)
        sl1 = zf
        for d in (labf * (p0 - tx), labf * (p1 - ty),
                  labf * (p2 - tw), labf * (p3 - th)):
            ad = jnp.abs(d)
            sl1 = sl1 + jnp.where(ad < 1.0, 0.5 * ad * ad, ad - 0.5)
        return reg_acc + sl1 * actf

    reg_acc = lax.fori_loop(0, ngrp, loss_body, zf)

    cntst_l[...] = cntv
    pltpu.sync_copy(cntst_l, sh_cnt.at[pl.ds(wid * 16, 16)])
    part_l[...] = reg_acc
    pltpu.sync_copy(part_l, sh_part.at[pl.ds(wid * 16, 16)])
    plsc.subcore_barrier()
    pltpu.sync_copy(sh_cnt, cntall_l)
    cnts = plsc.load_gather(cntall_l, [iota * 16])
    offset = jnp.sum(jnp.where(iota < wid, cnts, 0))

    def pos_body(q, _):
        rr = q * 16 + iota
        pos_l[pl.ds(q * 16, 16)] = jnp.where(rr < cnt_s, offset + rr,
                                             TOT_K + rr)
        return 0

    lax.fori_loop(0, 8, pos_body, 0)
    pltpu.sync_copy(gidx_l, sh_gidx.at[pos_l])
    pltpu.sync_copy(tcls_l, sh_tcls.at[pos_l])
    plsc.subcore_barrier()

    @pl.when(wid == 0)
    def _():
        pltpu.sync_copy(sh_gidx.at[pl.ds(0, TOT_K)], gidx_hbm)
        pltpu.sync_copy(sh_tcls.at[pl.ds(0, TOT_K)], tcls_hbm)
        pltpu.sync_copy(sh_part, partall_l)
        rs = zf
        for w in range(NSB):
            rs = rs + partall_l[pl.ds(w * 16, 16)]
        reg_t = jnp.sum(rs)
        out_l[...] = jnp.where(iota == 0, (REG_W / TOT_K) * reg_t, 0.0)
        pltpu.sync_copy(out_l, regv_hbm)


_call_b = pl.kernel(
    _body_b,
    out_type=(
        jax.ShapeDtypeStruct((TOT_K,), jnp.int32),
        jax.ShapeDtypeStruct((TOT_K,), jnp.int32),
        jax.ShapeDtypeStruct((16,), jnp.float32),
    ),
    mesh=_MESH_B,
    compiler_params=_SC_PARAMS,
    scratch_types=[
        pltpu.VMEM((NTB * 4,), jnp.float32),
        pltpu.VMEM((NTB * 4,), jnp.float32),
        pltpu.VMEM((G * 4,), jnp.float32),
        pltpu.VMEM((G,), jnp.int32),
        pltpu.VMEM((NTB,), jnp.int32),
        pltpu.VMEM((NSA * G,), jnp.int32),
        pltpu.VMEM((G,), jnp.int32),
        pltpu.VMEM((G,), jnp.int32),
        pltpu.VMEM((TOT_K,), jnp.int32),
        pltpu.VMEM((TOT_K,), jnp.int32),
        pltpu.VMEM((TOT_K,), jnp.int32),
        pltpu.VMEM((TOT_K,), jnp.int32),
        pltpu.VMEM((16,), jnp.int32),
        pltpu.VMEM((NSB * 16,), jnp.int32),
        pltpu.VMEM((16,), jnp.float32),
        pltpu.VMEM((NSB * 16,), jnp.float32),
        pltpu.VMEM((16,), jnp.float32),
        pltpu.VMEM_SHARED((2 * TOT_K,), jnp.int32),
        pltpu.VMEM_SHARED((2 * TOT_K,), jnp.int32),
        pltpu.VMEM_SHARED((NSB * 16,), jnp.int32),
        pltpu.VMEM_SHARED((NSB * 16,), jnp.float32),
    ],
)


def _body_c(scores_ref, gidx_s, gidx_v, tcls_v, regv_ref, out_ref,
            rows8_ref, sem):
    copies = []
    for r in range(TOT_K):
        tb = pl.multiple_of((gidx_s[r] >> 3) * 8, 8)
        cp = pltpu.make_async_copy(
            scores_ref.at[pl.ds(tb, 8), :],
            rows8_ref.at[pl.ds(r * 8, 8), :],
            sem,
        )
        cp.start()
        copies.append(cp)
    for cp in copies:
        cp.wait()

    rem_col = jnp.transpose((gidx_v[...] & 7).reshape(1, TOT_K))
    t_col = jnp.transpose(tcls_v[...].reshape(1, TOT_K))
    ci = jax.lax.broadcasted_iota(jnp.int32, (TOT_K, TOT_K * 8), 1)
    ri = jax.lax.broadcasted_iota(jnp.int32, (TOT_K, TOT_K * 8), 0)
    sel = (ci == ri * 8 + rem_col).astype(jnp.float32)
    rows = jax.lax.dot_general(
        sel, rows8_ref[...], (((1,), (0,)), ((), ())),
        preferred_element_type=jnp.float32)
    m = jnp.max(rows, axis=1, keepdims=True)
    lse = m + jnp.log(jnp.sum(jnp.exp(rows - m), axis=1, keepdims=True))
    onehot = jax.lax.broadcasted_iota(jnp.int32, (TOT_K, C), 1) == t_col
    logit_t = jnp.sum(jnp.where(onehot, rows, 0.0), axis=1, keepdims=True)
    cls_t = jnp.sum(logit_t - lse)
    reg = regv_ref[0]
    i2 = jax.lax.broadcasted_iota(jnp.int32, (8, 128), 1)
    r2 = jax.lax.broadcasted_iota(jnp.int32, (8, 128), 0)
    val = jnp.where(i2 == 0, -cls_t * (1.0 / TOT_K), reg)
    out_ref[...] = jnp.where((r2 == 0) & (i2 < 2), val, 0.0)


_call_c = pl.pallas_call(
    _body_c,
    out_shape=jax.ShapeDtypeStruct((8, 128), jnp.float32),
    in_specs=[
        pl.BlockSpec(memory_space=pltpu.HBM),
        pl.BlockSpec(memory_space=pltpu.SMEM),
        pl.BlockSpec(memory_space=pltpu.VMEM),
        pl.BlockSpec(memory_space=pltpu.VMEM),
        pl.BlockSpec(memory_space=pltpu.SMEM),
    ],
    out_specs=pl.BlockSpec(memory_space=pltpu.VMEM),
    scratch_shapes=[
        pltpu.VMEM((TOT_K * 8, C), jnp.float32),
        pltpu.SemaphoreType.DMA,
    ],
)


def kernel(rpn_proposals_bboxes, roi_score, roi_bboxes_txtytwth, gt_bboxes, gt_labels):
    rois_f = rpn_proposals_bboxes.reshape(-1)
    txty_f = roi_bboxes_txtytwth.reshape(-1)
    gt_f = gt_bboxes.reshape(-1)
    gtl = gt_labels.astype(jnp.int32)
    v, hist = _call_a(rois_f, gt_f)
    gidx, tcls, regv = _call_b(rois_f, txty_f, gt_f, gtl, v, hist)
    out = _call_c(roi_score, gidx, gidx, tcls, regv)
    return out[0, 0], out[0, 1]


# speedup vs baseline: 1.7895x; 1.0066x over previous
import jax
import jax.numpy as jnp
from jax import lax
from jax.experimental import pallas as pl
from jax.experimental.pallas import tpu as pltpu
from jax.experimental.pallas import tpu_sc as plsc

N = 20000
C = 81
G = 64
NSA = 32
NTA = 640
GRPSA = NTA // 16
NSB = 16
NTB = 1280
GRPSB = NTB // 16
POS_K = 32
NEG_K = 96
TOT_K = 128
REG_W = 2.0

_MESH_A = plsc.VectorSubcoreMesh(
    core_axis_name="c", subcore_axis_name="s", num_cores=2, num_subcores=16
)
_MESH_B = plsc.VectorSubcoreMesh(
    core_axis_name="c", subcore_axis_name="s", num_cores=1, num_subcores=16
)
_SC_PARAMS = pltpu.CompilerParams(needs_layout_passes=False)


def _ln(x):
    bits = lax.bitcast_convert_type(x, jnp.int32)
    e = jnp.right_shift(bits, 23) & 0xFF
    m = lax.bitcast_convert_type((bits & 0x7FFFFF) | 0x3F800000, jnp.float32)
    big = m > 1.4142135623730951
    m = jnp.where(big, m * 0.5, m)
    ef = (e - 127 + big.astype(jnp.int32)).astype(jnp.float32)
    s = (m - 1.0) / (m + 1.0)
    s2 = s * s
    p = jnp.float32(1.0 / 9.0)
    p = p * s2 + jnp.float32(1.0 / 7.0)
    p = p * s2 + jnp.float32(0.2)
    p = p * s2 + jnp.float32(1.0 / 3.0)
    p = p * s2 + jnp.float32(1.0)
    lnm = 2.0 * s * p
    return ef * jnp.float32(0.6931471805599453) + lnm


def _body_a(rois_hbm, gt_hbm, v_hbm, hist_hbm, rois_l, gt_l, areab_l,
            v_l, hist_l):
    wid = lax.axis_index("s") * 2 + lax.axis_index("c")
    own_lo = wid * NTA
    dbase = jnp.minimum(own_lo, N - NTA)
    iota = lax.iota(jnp.int32, 16)
    zc = jnp.zeros((16,), jnp.int32)

    pltpu.sync_copy(rois_hbm.at[pl.ds(dbase * 4, NTA * 4)], rois_l)
    pltpu.sync_copy(gt_hbm, gt_l)

    for q in range(4):
        hist_l[pl.ds(q * 16, 16)] = zc
        gidx16 = (q * 16 + iota) * 4
        bx0 = plsc.load_gather(gt_l, [gidx16])
        by0 = plsc.load_gather(gt_l, [gidx16 + 1])
        bx1 = plsc.load_gather(gt_l, [gidx16 + 2])
        by1 = plsc.load_gather(gt_l, [gidx16 + 3])
        areab_l[pl.ds(q * 16, 16)] = (bx1 - bx0) * (by1 - by0)

    @plsc.parallel_loop(0, GRPSA, unroll=2)
    def group_body(g):
        ridx = (g * 16 + iota) * 4
        ax0 = plsc.load_gather(rois_l, [ridx])
        ay0 = plsc.load_gather(rois_l, [ridx + 1])
        ax1 = plsc.load_gather(rois_l, [ridx + 2])
        ay1 = plsc.load_gather(rois_l, [ridx + 3])
        area_a = (ax1 - ax0) * (ay1 - ay0)

        def one_gt(j, best, bidx):
            bx0 = plsc.load_gather(gt_l, [zc + j * 4])
            by0 = plsc.load_gather(gt_l, [zc + (j * 4 + 1)])
            bx1 = plsc.load_gather(gt_l, [zc + (j * 4 + 2)])
            by1 = plsc.load_gather(gt_l, [zc + (j * 4 + 3)])
            ab = plsc.load_gather(areab_l, [zc + j])
            w = jnp.maximum(jnp.minimum(ax1, bx1) - jnp.maximum(ax0, bx0), 0.0)
            h = jnp.maximum(jnp.minimum(ay1, by1) - jnp.maximum(ay0, by0), 0.0)
            inter = w * h
            iou = inter / (area_a + ab - inter + 1e-8)
            upd = iou > best
            return jnp.where(upd, iou, best), jnp.where(upd, j, bidx)

        def gt_body(jj, carry):
            b1, i1, b2, i2 = carry
            for u in range(4):
                j = jj * 4 + u
                b1, i1 = one_gt(j, b1, i1)
                b2, i2 = one_gt(j + 32, b2, i2)
            return b1, i1, b2, i2

        neg1 = jnp.full((16,), -1.0, jnp.float32)
        b1, i1, b2, i2 = lax.fori_loop(
            0, 8, gt_body, (neg1, zc, neg1, zc)
        )
        upd = b2 > b1
        bidx = jnp.where(upd, i2, i1)
        v_l[pl.ds(g * 16, 16)] = bidx
        gi = dbase + g * 16 + iota
        valid = jnp.logical_and(gi >= own_lo, gi < N)
        plsc.addupdate_scatter(hist_l, [bidx], zc + 1, mask=valid)

    pltpu.sync_copy(v_l, v_hbm.at[pl.ds(dbase, NTA)])
    pltpu.sync_copy(hist_l, hist_hbm.at[pl.ds(wid * G, G)])


_call_a = pl.kernel(
    _body_a,
    out_type=(
        jax.ShapeDtypeStruct((N,), jnp.int32),
        jax.ShapeDtypeStruct((NSA * G,), jnp.int32),
    ),
    mesh=_MESH_A,
    compiler_params=_SC_PARAMS,
    scratch_types=[
        pltpu.VMEM((NTA * 4,), jnp.float32),
        pltpu.VMEM((G * 4,), jnp.float32),
        pltpu.VMEM((G,), jnp.float32),
        pltpu.VMEM((NTA,), jnp.int32),
        pltpu.VMEM((G,), jnp.int32),
    ],
)


def _body_b(rois_hbm, txty_hbm, gt_hbm, gtl_hbm, v_hbm, hist_hbm,
            gidx_hbm, tcls_hbm, regv_hbm,
            rois_l, txty_l, gt_l, gtl_l, v_l, histall_l, gsum_l, cdf_l,
            sel_l, gidx_l, tcls_l, pos_l, cntst_l, cntall_l, part_l,
            partall_l, out_l,
            sh_gidx, sh_tcls, sh_cnt, sh_part):
    wid = lax.axis_index("s")
    own_lo = wid * NTB
    dbase = jnp.minimum(own_lo, N - NTB)
    iota = lax.iota(jnp.int32, 16)
    zc = jnp.zeros((16,), jnp.int32)
    zf = jnp.zeros((16,), jnp.float32)

    pltpu.sync_copy(rois_hbm.at[pl.ds(dbase * 4, NTB * 4)], rois_l)
    pltpu.sync_copy(txty_hbm.at[pl.ds(dbase * 4, NTB * 4)], txty_l)
    pltpu.sync_copy(gt_hbm, gt_l)
    pltpu.sync_copy(gtl_hbm, gtl_l)
    pltpu.sync_copy(v_hbm.at[pl.ds(dbase, NTB)], v_l)
    pltpu.sync_copy(hist_hbm, histall_l)

    gq = []
    for q in range(4):
        acc = zc
        for w in range(NSA):
            acc = acc + histall_l[pl.ds(w * G + q * 16, 16)]
        gsum_l[pl.ds(q * 16, 16)] = acc
        gq.append(acc)

    cq = []
    tot = jnp.int32(0)
    for q in range(4):
        cc = plsc.cumsum(gq[q]) + tot
        tot = tot + jnp.sum(gq[q])
        cdf_l[pl.ds(q * 16, 16)] = cc
        cq.append(cc)

    npos = zc
    nneg = zc
    for q in range(4):
        cprev = cq[q] - gq[q]
        npos = npos + plsc.all_reduce_population_count(cprev <= N - POS_K)
        nneg = nneg + plsc.all_reduce_population_count(cq[q] < NEG_K)
    tpos = jnp.max(npos) - 1
    tneg = jnp.max(nneg)
    cpos = jnp.max(plsc.load_gather(cdf_l, [zc + tpos]))
    rpos = POS_K - (N - cpos)
    cneg = jnp.max(plsc.load_gather(cdf_l, [zc + tneg]))
    gneg = jnp.max(plsc.load_gather(gsum_l, [zc + tneg]))
    rneg = NEG_K - (cneg - gneg)

    hp0 = plsc.load_gather(histall_l, [iota * G + tpos])
    hp1 = plsc.load_gather(histall_l, [(iota + 16) * G + tpos])
    hn0 = plsc.load_gather(histall_l, [iota * G + tneg])
    hn1 = plsc.load_gather(histall_l, [(iota + 16) * G + tneg])
    a2 = wid * 2
    base_pos = (jnp.sum(jnp.where(iota < a2, hp0, 0))
                + jnp.sum(jnp.where(iota + 16 < a2, hp1, 0)))
    base_neg = (jnp.sum(jnp.where(iota < a2, hn0, 0))
                + jnp.sum(jnp.where(iota + 16 < a2, hn1, 0)))

    for q in range(8):
        sel_l[pl.ds(q * 16, 16)] = zc

    def sel_body(g, carry):
        cntv, tpv, tnv = carry
        v = v_l[pl.ds(g * 16, 16)]
        gi = dbase + g * 16 + iota
        valid = jnp.logical_and(gi >= own_lo, gi < N)
        hi = jnp.logical_and(v > tpos, valid)
        mtp = jnp.logical_and(v == tpos, valid)
        rkp = tpv + plsc.cumsum(mtp.astype(jnp.int32)) - 1 + base_pos
        ptie = jnp.logical_and(mtp, rkp < rpos)
        lo = jnp.logical_and(v < tneg, valid)
        mtn = jnp.logical_and(v == tneg, valid)
        rkn = tnv + plsc.cumsum(mtn.astype(jnp.int32)) - 1 + base_neg
        ntie = jnp.logical_and(mtn, rkn < rneg)
        sel = jnp.logical_or(jnp.logical_or(hi, ptie), jnp.logical_or(lo, ntie))
        pos = cntv + plsc.cumsum(sel.astype(jnp.int32)) - 1
        plsc.store_scatter(sel_l, [pos], g * 16 + iota, mask=sel)
        cntv = cntv + plsc.all_reduce_population_count(sel)
        tpv = tpv + plsc.all_reduce_population_count(mtp)
        tnv = tnv + plsc.all_reduce_population_count(mtn)
        return cntv, tpv, tnv

    cntv, _, _ = lax.fori_loop(0, GRPSB, sel_body, (zc, zc, zc))
    cnt_s = jnp.max(cntv)

    ngrp = jnp.right_shift(cnt_s + 15, 4)

    def loss_body(q, reg_acc):
        rvec = q * 16 + iota
        actf = (rvec < cnt_s).astype(jnp.float32)
        lidx = plsc.load_gather(sel_l, [rvec])
        vr = plsc.load_gather(v_l, [lidx])
        lab = (vr >= 1).astype(jnp.int32)
        labf = lab.astype(jnp.float32)
        gl = plsc.load_gather(gtl_l, [vr])
        tcls_l[pl.ds(q * 16, 16)] = jnp.clip(gl * lab, 0, C - 1)
        gidx_l[pl.ds(q * 16, 16)] = lidx + dbase

        l4 = lidx * 4
        v4 = vr * 4
        ax0 = plsc.load_gather(rois_l, [l4])
        ay0 = plsc.load_gather(rois_l, [l4 + 1])
        ax1 = plsc.load_gather(rois_l, [l4 + 2])
        ay1 = plsc.load_gather(rois_l, [l4 + 3])
        p0 = plsc.load_gather(txty_l, [l4])
        p1 = plsc.load_gather(txty_l, [l4 + 1])
        p2 = plsc.load_gather(txty_l, [l4 + 2])
        p3 = plsc.load_gather(txty_l, [l4 + 3])
        gx0 = plsc.load_gather(gt_l, [v4])
        gy0 = plsc.load_gather(gt_l, [v4 + 1])
        gx1 = plsc.load_gather(gt_l, [v4 + 2])
        gy1 = plsc.load_gather(gt_l, [v4 + 3])
        aw = ax1 - ax0
        ah = ay1 - ay0
        axc = ax0 + 0.5 * aw
        ayc = ay0 + 0.5 * ah
        gw = gx1 - gx0
        gh = gy1 - gy0
        gxc = gx0 + 0.5 * gw
        gyc = gy0 + 0.5 * gh
        awm = jnp.maximum(aw, 1e-8)
        ahm = jnp.maximum(ah, 1e-8)
        tx = (gxc - axc) / awm
        ty = (gyc - ayc) / ahm
        tw = _ln(jnp.maximum(gw, 1e-8) / awm)
        th = _ln(jnp.maximum(gh, 1e-8) / ahm)
        sl1 = zf
        for d in (labf * (p0 - tx), labf * (p1 - ty),
                  labf * (p2 - tw), labf * (p3 - th)):
            ad = jnp.abs(d)
            sl1 = sl1 + jnp.where(ad < 1.0, 0.5 * ad * ad, ad - 0.5)
        return reg_acc + sl1 * actf

    reg_acc = lax.fori_loop(0, ngrp, loss_body, zf)

    cntst_l[...] = cntv
    pltpu.sync_copy(cntst_l, sh_cnt.at[pl.ds(wid * 16, 16)])
    part_l[...] = reg_acc
    pltpu.sync_copy(part_l, sh_part.at[pl.ds(wid * 16, 16)])
    plsc.subcore_barrier()
    pltpu.sync_copy(sh_cnt, cntall_l)
    cnts = plsc.load_gather(cntall_l, [iota * 16])
    offset = jnp.sum(jnp.where(iota < wid, cnts, 0))

    def pos_body(q, _):
        rr = q * 16 + iota
        pos_l[pl.ds(q * 16, 16)] = jnp.where(rr < cnt_s, offset + rr,
                                             TOT_K + rr)
        return 0

    lax.fori_loop(0, 8, pos_body, 0)
    pltpu.sync_copy(gidx_l, sh_gidx.at[pos_l])
    pltpu.sync_copy(tcls_l, sh_tcls.at[pos_l])
    plsc.subcore_barrier()

    @pl.when(wid == 0)
    def _():
        pltpu.sync_copy(sh_gidx.at[pl.ds(0, TOT_K)], gidx_hbm)
        pltpu.sync_copy(sh_tcls.at[pl.ds(0, TOT_K)], tcls_hbm)
        pltpu.sync_copy(sh_part, partall_l)
        rs = zf
        for w in range(NSB):
            rs = rs + partall_l[pl.ds(w * 16, 16)]
        reg_t = jnp.sum(rs)
        out_l[...] = jnp.where(iota == 0, (REG_W / TOT_K) * reg_t, 0.0)
        pltpu.sync_copy(out_l, regv_hbm)


_call_b = pl.kernel(
    _body_b,
    out_type=(
        jax.ShapeDtypeStruct((TOT_K,), jnp.int32),
        jax.ShapeDtypeStruct((TOT_K,), jnp.int32),
        jax.ShapeDtypeStruct((16,), jnp.float32),
    ),
    mesh=_MESH_B,
    compiler_params=_SC_PARAMS,
    scratch_types=[
        pltpu.VMEM((NTB * 4,), jnp.float32),
        pltpu.VMEM((NTB * 4,), jnp.float32),
        pltpu.VMEM((G * 4,), jnp.float32),
        pltpu.VMEM((G,), jnp.int32),
        pltpu.VMEM((NTB,), jnp.int32),
        pltpu.VMEM((NSA * G,), jnp.int32),
        pltpu.VMEM((G,), jnp.int32),
        pltpu.VMEM((G,), jnp.int32),
        pltpu.VMEM((TOT_K,), jnp.int32),
        pltpu.VMEM((TOT_K,), jnp.int32),
        pltpu.VMEM((TOT_K,), jnp.int32),
        pltpu.VMEM((TOT_K,), jnp.int32),
        pltpu.VMEM((16,), jnp.int32),
        pltpu.VMEM((NSB * 16,), jnp.int32),
        pltpu.VMEM((16,), jnp.float32),
        pltpu.VMEM((NSB * 16,), jnp.float32),
        pltpu.VMEM((16,), jnp.float32),
        pltpu.VMEM_SHARED((2 * TOT_K,), jnp.int32),
        pltpu.VMEM_SHARED((2 * TOT_K,), jnp.int32),
        pltpu.VMEM_SHARED((NSB * 16,), jnp.int32),
        pltpu.VMEM_SHARED((NSB * 16,), jnp.float32),
    ],
)


def _body_c(scores_ref, gidx_s, gidx_v, tcls_v, regv_ref, out_ref,
            rows8_ref, sem):
    copies = []
    for r in range(TOT_K):
        tb = pl.multiple_of((gidx_s[r] >> 3) * 8, 8)
        cp = pltpu.make_async_copy(
            scores_ref.at[pl.ds(tb, 8), :],
            rows8_ref.at[pl.ds(r * 8, 8), :],
            sem,
        )
        cp.start()
        copies.append(cp)
    for cp in copies:
        cp.wait()

    rem_col = jnp.transpose((gidx_v[...] & 7).reshape(1, TOT_K))
    t_col = jnp.transpose(tcls_v[...].reshape(1, TOT_K))
    ci = jax.lax.broadcasted_iota(jnp.int32, (TOT_K, TOT_K * 8), 1)
    ri = jax.lax.broadcasted_iota(jnp.int32, (TOT_K, TOT_K * 8), 0)
    sel = (ci == ri * 8 + rem_col).astype(jnp.float32)
    rows = jax.lax.dot_general(
        sel, rows8_ref[...], (((1,), (0,)), ((), ())),
        preferred_element_type=jnp.float32)
    m = jnp.max(rows, axis=1, keepdims=True)
    lse = m + jnp.log(jnp.sum(jnp.exp(rows - m), axis=1, keepdims=True))
    onehot = jax.lax.broadcasted_iota(jnp.int32, (TOT_K, C), 1) == t_col
    logit_t = jnp.sum(jnp.where(onehot, rows, 0.0), axis=1, keepdims=True)
    cls_t = jnp.sum(logit_t - lse)
    reg = regv_ref[0]
    i2 = jax.lax.broadcasted_iota(jnp.int32, (8, 128), 1)
    r2 = jax.lax.broadcasted_iota(jnp.int32, (8, 128), 0)
    val = jnp.where(i2 == 0, -cls_t * (1.0 / TOT_K), reg)
    out_ref[...] = jnp.where((r2 == 0) & (i2 < 2), val, 0.0)


_call_c = pl.pallas_call(
    _body_c,
    out_shape=jax.ShapeDtypeStruct((8, 128), jnp.float32),
    in_specs=[
        pl.BlockSpec(memory_space=pltpu.HBM),
        pl.BlockSpec(memory_space=pltpu.SMEM),
        pl.BlockSpec(memory_space=pltpu.VMEM),
        pl.BlockSpec(memory_space=pltpu.VMEM),
        pl.BlockSpec(memory_space=pltpu.SMEM),
    ],
    out_specs=pl.BlockSpec(memory_space=pltpu.VMEM),
    scratch_shapes=[
        pltpu.VMEM((TOT_K * 8, C), jnp.float32),
        pltpu.SemaphoreType.DMA,
    ],
)


def kernel(rpn_proposals_bboxes, roi_score, roi_bboxes_txtytwth, gt_bboxes, gt_labels):
    rois_f = rpn_proposals_bboxes.reshape(-1)
    txty_f = roi_bboxes_txtytwth.reshape(-1)
    gt_f = gt_bboxes.reshape(-1)
    gtl = gt_labels.astype(jnp.int32)
    v, hist = _call_a(rois_f, gt_f)
    gidx, tcls, regv = _call_b(rois_f, txty_f, gt_f, gtl, v, hist)
    out = _call_c(roi_score, gidx, gidx, tcls, regv)
    return out[0, 0], out[0, 1]


# speedup vs baseline: 1.8343x; 1.0251x over previous
import jax
import jax.numpy as jnp
from jax import lax
from jax.experimental import pallas as pl
from jax.experimental.pallas import tpu as pltpu
from jax.experimental.pallas import tpu_sc as plsc

N = 20000
C = 81
G = 64
NSA = 32
NTA = 640
GRPSA = NTA // 16
NSB = 16
NTB = 1280
GRPSB = NTB // 16
POS_K = 32
NEG_K = 96
TOT_K = 128
REG_W = 2.0

_MESH_A = plsc.VectorSubcoreMesh(
    core_axis_name="c", subcore_axis_name="s", num_cores=2, num_subcores=16
)
_MESH_B = plsc.VectorSubcoreMesh(
    core_axis_name="c", subcore_axis_name="s", num_cores=1, num_subcores=16
)
_SC_PARAMS = pltpu.CompilerParams(needs_layout_passes=False)


def _ln(x):
    bits = lax.bitcast_convert_type(x, jnp.int32)
    e = jnp.right_shift(bits, 23) & 0xFF
    m = lax.bitcast_convert_type((bits & 0x7FFFFF) | 0x3F800000, jnp.float32)
    big = m > 1.4142135623730951
    m = jnp.where(big, m * 0.5, m)
    ef = (e - 127 + big.astype(jnp.int32)).astype(jnp.float32)
    s = (m - 1.0) / (m + 1.0)
    s2 = s * s
    p = jnp.float32(1.0 / 9.0)
    p = p * s2 + jnp.float32(1.0 / 7.0)
    p = p * s2 + jnp.float32(0.2)
    p = p * s2 + jnp.float32(1.0 / 3.0)
    p = p * s2 + jnp.float32(1.0)
    lnm = 2.0 * s * p
    return ef * jnp.float32(0.6931471805599453) + lnm


def _body_a(rois_hbm, gt_hbm, v_hbm, hist_hbm, rois_l, gt_l, areab_l,
            v_l, hist_l):
    wid = lax.axis_index("s") * 2 + lax.axis_index("c")
    own_lo = wid * NTA
    dbase = jnp.minimum(own_lo, N - NTA)
    iota = lax.iota(jnp.int32, 16)
    zc = jnp.zeros((16,), jnp.int32)

    pltpu.sync_copy(rois_hbm.at[pl.ds(dbase * 4, NTA * 4)], rois_l)
    pltpu.sync_copy(gt_hbm, gt_l)

    for q in range(4):
        hist_l[pl.ds(q * 16, 16)] = zc
        gidx16 = (q * 16 + iota) * 4
        bx0 = plsc.load_gather(gt_l, [gidx16])
        by0 = plsc.load_gather(gt_l, [gidx16 + 1])
        bx1 = plsc.load_gather(gt_l, [gidx16 + 2])
        by1 = plsc.load_gather(gt_l, [gidx16 + 3])
        areab_l[pl.ds(q * 16, 16)] = (bx1 - bx0) * (by1 - by0)

    @plsc.parallel_loop(0, GRPSA, unroll=2)
    def group_body(g):
        ridx = (g * 16 + iota) * 4
        ax0 = plsc.load_gather(rois_l, [ridx])
        ay0 = plsc.load_gather(rois_l, [ridx + 1])
        ax1 = plsc.load_gather(rois_l, [ridx + 2])
        ay1 = plsc.load_gather(rois_l, [ridx + 3])
        area_a = (ax1 - ax0) * (ay1 - ay0)

        def one_gt(j, best, bidx):
            bx0 = plsc.load_gather(gt_l, [zc + j * 4])
            by0 = plsc.load_gather(gt_l, [zc + (j * 4 + 1)])
            bx1 = plsc.load_gather(gt_l, [zc + (j * 4 + 2)])
            by1 = plsc.load_gather(gt_l, [zc + (j * 4 + 3)])
            ab = plsc.load_gather(areab_l, [zc + j])
            w = jnp.maximum(jnp.minimum(ax1, bx1) - jnp.maximum(ax0, bx0), 0.0)
            h = jnp.maximum(jnp.minimum(ay1, by1) - jnp.maximum(ay0, by0), 0.0)
            inter = w * h
            iou = inter / (area_a + ab - inter + 1e-8)
            upd = iou > best
            return jnp.where(upd, iou, best), jnp.where(upd, j, bidx)

        def gt_body(jj, carry):
            b1, i1, b2, i2 = carry
            for u in range(4):
                j = jj * 4 + u
                b1, i1 = one_gt(j, b1, i1)
                b2, i2 = one_gt(j + 32, b2, i2)
            return b1, i1, b2, i2

        neg1 = jnp.full((16,), -1.0, jnp.float32)
        b1, i1, b2, i2 = lax.fori_loop(
            0, 8, gt_body, (neg1, zc, neg1, zc)
        )
        upd = b2 > b1
        bidx = jnp.where(upd, i2, i1)
        v_l[pl.ds(g * 16, 16)] = bidx
        gi = dbase + g * 16 + iota
        valid = jnp.logical_and(gi >= own_lo, gi < N)
        plsc.addupdate_scatter(hist_l, [bidx], zc + 1, mask=valid)

    pltpu.sync_copy(v_l, v_hbm.at[pl.ds(dbase, NTA)])
    pltpu.sync_copy(hist_l, hist_hbm.at[pl.ds(wid * G, G)])


_call_a = pl.kernel(
    _body_a,
    out_type=(
        jax.ShapeDtypeStruct((N,), jnp.int32),
        jax.ShapeDtypeStruct((NSA * G,), jnp.int32),
    ),
    mesh=_MESH_A,
    compiler_params=_SC_PARAMS,
    scratch_types=[
        pltpu.VMEM((NTA * 4,), jnp.float32),
        pltpu.VMEM((G * 4,), jnp.float32),
        pltpu.VMEM((G,), jnp.float32),
        pltpu.VMEM((NTA,), jnp.int32),
        pltpu.VMEM((G,), jnp.int32),
    ],
)


def _body_b(gtl_hbm, v_hbm, hist_hbm,
            gidx_hbm, tcls_hbm, vsel_hbm,
            gtl_l, v_l, histall_l, gsum_l, cdf_l,
            sel_l, gidx_l, tcls_l, vsel_l, pos_l, cntst_l, cntall_l,
            sh_gidx, sh_tcls, sh_vsel, sh_cnt):
    wid = lax.axis_index("s")
    own_lo = wid * NTB
    dbase = jnp.minimum(own_lo, N - NTB)
    iota = lax.iota(jnp.int32, 16)
    zc = jnp.zeros((16,), jnp.int32)

    pltpu.sync_copy(gtl_hbm, gtl_l)
    pltpu.sync_copy(v_hbm.at[pl.ds(dbase, NTB)], v_l)
    pltpu.sync_copy(hist_hbm, histall_l)

    gq = []
    for q in range(4):
        acc = zc
        for w in range(NSA):
            acc = acc + histall_l[pl.ds(w * G + q * 16, 16)]
        gsum_l[pl.ds(q * 16, 16)] = acc
        gq.append(acc)

    cq = []
    tot = jnp.int32(0)
    for q in range(4):
        cc = plsc.cumsum(gq[q]) + tot
        tot = tot + jnp.sum(gq[q])
        cdf_l[pl.ds(q * 16, 16)] = cc
        cq.append(cc)

    npos = zc
    nneg = zc
    for q in range(4):
        cprev = cq[q] - gq[q]
        npos = npos + plsc.all_reduce_population_count(cprev <= N - POS_K)
        nneg = nneg + plsc.all_reduce_population_count(cq[q] < NEG_K)
    tpos = jnp.max(npos) - 1
    tneg = jnp.max(nneg)
    cpos = jnp.max(plsc.load_gather(cdf_l, [zc + tpos]))
    rpos = POS_K - (N - cpos)
    cneg = jnp.max(plsc.load_gather(cdf_l, [zc + tneg]))
    gneg = jnp.max(plsc.load_gather(gsum_l, [zc + tneg]))
    rneg = NEG_K - (cneg - gneg)

    hp0 = plsc.load_gather(histall_l, [iota * G + tpos])
    hp1 = plsc.load_gather(histall_l, [(iota + 16) * G + tpos])
    hn0 = plsc.load_gather(histall_l, [iota * G + tneg])
    hn1 = plsc.load_gather(histall_l, [(iota + 16) * G + tneg])
    a2 = wid * 2
    base_pos = (jnp.sum(jnp.where(iota < a2, hp0, 0))
                + jnp.sum(jnp.where(iota + 16 < a2, hp1, 0)))
    base_neg = (jnp.sum(jnp.where(iota < a2, hn0, 0))
                + jnp.sum(jnp.where(iota + 16 < a2, hn1, 0)))

    for q in range(8):
        sel_l[pl.ds(q * 16, 16)] = zc

    def sel_body(g, carry):
        cntv, tpv, tnv = carry
        v = v_l[pl.ds(g * 16, 16)]
        gi = dbase + g * 16 + iota
        valid = jnp.logical_and(gi >= own_lo, gi < N)
        hi = jnp.logical_and(v > tpos, valid)
        mtp = jnp.logical_and(v == tpos, valid)
        rkp = tpv + plsc.cumsum(mtp.astype(jnp.int32)) - 1 + base_pos
        ptie = jnp.logical_and(mtp, rkp < rpos)
        lo = jnp.logical_and(v < tneg, valid)
        mtn = jnp.logical_and(v == tneg, valid)
        rkn = tnv + plsc.cumsum(mtn.astype(jnp.int32)) - 1 + base_neg
        ntie = jnp.logical_and(mtn, rkn < rneg)
        sel = jnp.logical_or(jnp.logical_or(hi, ptie), jnp.logical_or(lo, ntie))
        pos = cntv + plsc.cumsum(sel.astype(jnp.int32)) - 1
        plsc.store_scatter(sel_l, [pos], g * 16 + iota, mask=sel)
        cntv = cntv + plsc.all_reduce_population_count(sel)
        tpv = tpv + plsc.all_reduce_population_count(mtp)
        tnv = tnv + plsc.all_reduce_population_count(mtn)
        return cntv, tpv, tnv

    cntv, _, _ = lax.fori_loop(0, GRPSB, sel_body, (zc, zc, zc))
    cnt_s = jnp.max(cntv)

    ngrp = jnp.right_shift(cnt_s + 15, 4)

    def loss_body(q, _):
        rvec = q * 16 + iota
        lidx = plsc.load_gather(sel_l, [rvec])
        vr = plsc.load_gather(v_l, [lidx])
        lab = (vr >= 1).astype(jnp.int32)
        gl = plsc.load_gather(gtl_l, [vr])
        tcls_l[pl.ds(q * 16, 16)] = jnp.clip(gl * lab, 0, C - 1)
        gidx_l[pl.ds(q * 16, 16)] = lidx + dbase
        vsel_l[pl.ds(q * 16, 16)] = vr
        return 0

    lax.fori_loop(0, ngrp, loss_body, 0)

    cntst_l[...] = cntv
    pltpu.sync_copy(cntst_l, sh_cnt.at[pl.ds(wid * 16, 16)])
    plsc.subcore_barrier()
    pltpu.sync_copy(sh_cnt, cntall_l)
    cnts = plsc.load_gather(cntall_l, [iota * 16])
    offset = jnp.sum(jnp.where(iota < wid, cnts, 0))

    def pos_body(q, _):
        rr = q * 16 + iota
        pos_l[pl.ds(q * 16, 16)] = jnp.where(rr < cnt_s, offset + rr,
                                             TOT_K + rr)
        return 0

    lax.fori_loop(0, 8, pos_body, 0)
    pltpu.sync_copy(gidx_l, sh_gidx.at[pos_l])
    pltpu.sync_copy(tcls_l, sh_tcls.at[pos_l])
    pltpu.sync_copy(vsel_l, sh_vsel.at[pos_l])
    plsc.subcore_barrier()

    @pl.when(wid == 0)
    def _():
        pltpu.sync_copy(sh_gidx.at[pl.ds(0, TOT_K)], gidx_hbm)
        pltpu.sync_copy(sh_tcls.at[pl.ds(0, TOT_K)], tcls_hbm)
        pltpu.sync_copy(sh_vsel.at[pl.ds(0, TOT_K)], vsel_hbm)


_call_b = pl.kernel(
    _body_b,
    out_type=(
        jax.ShapeDtypeStruct((TOT_K,), jnp.int32),
        jax.ShapeDtypeStruct((TOT_K,), jnp.int32),
        jax.ShapeDtypeStruct((TOT_K,), jnp.int32),
    ),
    mesh=_MESH_B,
    compiler_params=_SC_PARAMS,
    scratch_types=[
        pltpu.VMEM((G,), jnp.int32),
        pltpu.VMEM((NTB,), jnp.int32),
        pltpu.VMEM((NSA * G,), jnp.int32),
        pltpu.VMEM((G,), jnp.int32),
        pltpu.VMEM((G,), jnp.int32),
        pltpu.VMEM((TOT_K,), jnp.int32),
        pltpu.VMEM((TOT_K,), jnp.int32),
        pltpu.VMEM((TOT_K,), jnp.int32),
        pltpu.VMEM((TOT_K,), jnp.int32),
        pltpu.VMEM((TOT_K,), jnp.int32),
        pltpu.VMEM((16,), jnp.int32),
        pltpu.VMEM((NSB * 16,), jnp.int32),
        pltpu.VMEM_SHARED((2 * TOT_K,), jnp.int32),
        pltpu.VMEM_SHARED((2 * TOT_K,), jnp.int32),
        pltpu.VMEM_SHARED((2 * TOT_K,), jnp.int32),
        pltpu.VMEM_SHARED((NSB * 16,), jnp.int32),
    ],
)


def _body_c(scores_ref, rpn_ref, txty_ref, gt_ref, gidx_s, gidx_v, tcls_v,
            vsel_v, out_ref, rows8_ref, box8_ref, prd8_ref, sem):
    copies = []
    for r in range(TOT_K):
        tb = pl.multiple_of((gidx_s[r] >> 3) * 8, 8)
        for src, dst in ((scores_ref, rows8_ref), (rpn_ref, box8_ref),
                         (txty_ref, prd8_ref)):
            cp = pltpu.make_async_copy(
                src.at[pl.ds(tb, 8), :], dst.at[pl.ds(r * 8, 8), :], sem)
            cp.start()
            copies.append(cp)
    for cp in copies:
        cp.wait()

    rem_col = jnp.transpose((gidx_v[...] & 7).reshape(1, TOT_K))
    t_col = jnp.transpose(tcls_v[...].reshape(1, TOT_K))
    v_col = jnp.transpose(vsel_v[...].reshape(1, TOT_K))
    ci = jax.lax.broadcasted_iota(jnp.int32, (TOT_K, TOT_K * 8), 1)
    ri = jax.lax.broadcasted_iota(jnp.int32, (TOT_K, TOT_K * 8), 0)
    sel = (ci == ri * 8 + rem_col).astype(jnp.float32)
    dn = (((1,), (0,)), ((), ()))
    hi_p = jax.lax.Precision.HIGHEST
    rows = jax.lax.dot_general(
        sel, rows8_ref[...], dn, precision=hi_p, preferred_element_type=jnp.float32)
    a = jax.lax.dot_general(
        sel, box8_ref[...], dn, precision=hi_p, preferred_element_type=jnp.float32)
    p = jax.lax.dot_general(
        sel, prd8_ref[...], dn, precision=hi_p, preferred_element_type=jnp.float32)
    gsel = (jax.lax.broadcasted_iota(jnp.int32, (TOT_K, G), 1)
            == v_col).astype(jnp.float32)
    g = jax.lax.dot_general(
        gsel, gt_ref[...], dn, precision=hi_p, preferred_element_type=jnp.float32)

    m = jnp.max(rows, axis=1, keepdims=True)
    lse = m + jnp.log(jnp.sum(jnp.exp(rows - m), axis=1, keepdims=True))
    onehot = jax.lax.broadcasted_iota(jnp.int32, (TOT_K, C), 1) == t_col
    logit_t = jnp.sum(jnp.where(onehot, rows, 0.0), axis=1, keepdims=True)
    cls_t = jnp.sum(logit_t - lse)

    labf = (v_col >= 1).astype(jnp.float32)
    aw = a[:, 2:3] - a[:, 0:1]
    ah = a[:, 3:4] - a[:, 1:2]
    axc = a[:, 0:1] + 0.5 * aw
    ayc = a[:, 1:2] + 0.5 * ah
    gw = g[:, 2:3] - g[:, 0:1]
    gh = g[:, 3:4] - g[:, 1:2]
    gxc = g[:, 0:1] + 0.5 * gw
    gyc = g[:, 1:2] + 0.5 * gh
    awm = jnp.maximum(aw, 1e-8)
    ahm = jnp.maximum(ah, 1e-8)
    tx = (gxc - axc) / awm
    ty = (gyc - ayc) / ahm
    tw = jnp.log(jnp.maximum(gw, 1e-8) / awm)
    th = jnp.log(jnp.maximum(gh, 1e-8) / ahm)
    reg_t = jnp.float32(0.0)
    for d in (labf * (p[:, 0:1] - tx), labf * (p[:, 1:2] - ty),
              labf * (p[:, 2:3] - tw), labf * (p[:, 3:4] - th)):
        ad = jnp.abs(d)
        reg_t = reg_t + jnp.sum(jnp.where(ad < 1.0, 0.5 * ad * ad, ad - 0.5))

    i2 = jax.lax.broadcasted_iota(jnp.int32, (8, 128), 1)
    r2 = jax.lax.broadcasted_iota(jnp.int32, (8, 128), 0)
    val = jnp.where(i2 == 0, -cls_t * (1.0 / TOT_K), (REG_W / TOT_K) * reg_t)
    out_ref[...] = jnp.where((r2 == 0) & (i2 < 2), val, 0.0)


_call_c = pl.pallas_call(
    _body_c,
    out_shape=jax.ShapeDtypeStruct((8, 128), jnp.float32),
    in_specs=[
        pl.BlockSpec(memory_space=pltpu.HBM),
        pl.BlockSpec(memory_space=pltpu.HBM),
        pl.BlockSpec(memory_space=pltpu.HBM),
        pl.BlockSpec(memory_space=pltpu.VMEM),
        pl.BlockSpec(memory_space=pltpu.SMEM),
        pl.BlockSpec(memory_space=pltpu.VMEM),
        pl.BlockSpec(memory_space=pltpu.VMEM),
        pl.BlockSpec(memory_space=pltpu.VMEM),
    ],
    out_specs=pl.BlockSpec(memory_space=pltpu.VMEM),
    scratch_shapes=[
        pltpu.VMEM((TOT_K * 8, C), jnp.float32),
        pltpu.VMEM((TOT_K * 8, 4), jnp.float32),
        pltpu.VMEM((TOT_K * 8, 4), jnp.float32),
        pltpu.SemaphoreType.DMA,
    ],
)


def kernel(rpn_proposals_bboxes, roi_score, roi_bboxes_txtytwth, gt_bboxes, gt_labels):
    rois_f = rpn_proposals_bboxes.reshape(-1)
    gt_f = gt_bboxes.reshape(-1)
    gtl = gt_labels.astype(jnp.int32)
    v, hist = _call_a(rois_f, gt_f)
    gidx, tcls, vsel = _call_b(gtl, v, hist)
    out = _call_c(roi_score, rpn_proposals_bboxes, roi_bboxes_txtytwth,
                  gt_bboxes, gidx, gidx, tcls, vsel)
    return out[0, 0], out[0, 1]
